# baseline (speedup 1.0000x reference)
import jax
import jax.numpy as jnp
from jax import lax
from jax.experimental import pallas as pl
from jax.experimental.pallas import tpu as pltpu
from jax.experimental.pallas import tpu_sc as plsc

F32 = jnp.float32
I32 = jnp.int32
NC = 2
NS = 16
HID = 128
BR = 256
NBUF = 2
BLK = 128

_MESH = dict(core_axis_name="c", subcore_axis_name="s",
             num_cores=NC, num_subcores=NS)
_CPARAMS = dict(needs_layout_passes=False, use_tc_tiling_on_sc=False)


def _cdiv(a, b):
    return -(-a // b)


def _et_of(n_edges):
    return max(2, _cdiv(n_edges, NS * 128)) * 128


_SEG_CACHE = {}
_SPMEM_BUDGET = 4_700_000


def _seg_geometry(n_dst):
    p = 1
    while True:
        chunk = _cdiv(n_dst, NC * p * 128) * 128
        if (chunk + 128) * 512 <= _SPMEM_BUDGET:
            return p, chunk
        p += 1


_A_MAX = 8576


def _make_seg_sum(n_src, n_dst, n_edges):
    key = (n_src, n_dst, n_edges)
    if key in _SEG_CACHE:
        return _SEG_CACHE[key]

    et = _et_of(n_edges)
    P, chunk = _seg_geometry(n_dst)
    A = chunk + 128
    assert A <= _A_MAX
    n_out = NC * P * chunk
    zr = A // 16
    wr = chunk // 16

    scratch = [
        pltpu.VMEM((et,), I32),
        pltpu.VMEM((et,), I32),
        pltpu.VMEM((et + 16,), I32),
        pltpu.VMEM((et + 16,), I32),
        pltpu.VMEM((NBUF, BLK, HID), F32),
        pltpu.VMEM_SHARED((A, HID), F32),
    ]
    scratch += [pltpu.SemaphoreType.DMA] * (2 * NBUF + 1)

    mesh = plsc.VectorSubcoreMesh(**_MESH)

    def body(hsrc, src_hbm, dst_hbm, z_hbm, sums_o, src_raw, dst_raw,
             lsrc, ldst, rows, acc, *sems):
        gsem = sems[:NBUF]
        ssem = sems[NBUF:2 * NBUF]
        zsem = sems[2 * NBUF]

        c = lax.axis_index("c")
        s = lax.axis_index("s")

        base = s * et
        pltpu.sync_copy(src_hbm.at[pl.ds(base, et)], src_raw)
        pltpu.sync_copy(dst_hbm.at[pl.ds(base, et)], dst_raw)

        zb = s * zr

        def g_issue(j, b):
            pltpu.async_copy(
                hsrc.at[lsrc.at[pl.ds(j * BLK, BLK)]], rows.at[b], gsem[b])

        def g_wait(b):
            pltpu.make_async_copy(
                hsrc.at[lsrc.at[pl.ds(0, BLK)]], rows.at[b], gsem[b]).wait()

        def s_issue(j, b):
            pltpu.async_copy(rows.at[b],
                             acc.at[ldst.at[pl.ds(j * BLK, BLK)]],
                             ssem[b], add=True)

        def s_wait(b):
            pltpu.make_async_copy(
                rows.at[b], acc.at[ldst.at[pl.ds(0, BLK)]], ssem[b]).wait()

        for p in range(P):
            ri = c * P + p
            lo = ri * chunk

            pltpu.async_copy(z_hbm.at[pl.ds(0, zr)], acc.at[pl.ds(zb, zr)],
                             zsem)

            zivec = jnp.zeros((16,), I32)
            tvec = jnp.full((16,), chunk, I32)

            def fill(i, carry):
                lsrc[pl.ds(i * 16, 16)] = zivec
                ldst[pl.ds(i * 16, 16)] = tvec
                return carry

            lax.fori_loop(0, et // 16 + 1, fill, 0)

            def scan(g, off):
                d = dst_raw[pl.ds(g * 16, 16)]
                sv = src_raw[pl.ds(g * 16, 16)]
                m = (d >= lo) & (d < lo + chunk)
                plsc.store_compressed(lsrc.at[pl.ds(off, 16)], sv, mask=m)
                plsc.store_compressed(ldst.at[pl.ds(off, 16)], d - lo, mask=m)
                return off + jnp.max(plsc.all_reduce_population_count(m))

            m_cnt = lax.fori_loop(0, et // 16, scan, jnp.int32(0))
            nb = (m_cnt + BLK - 1) // BLK

            pltpu.make_async_copy(z_hbm.at[pl.ds(0, zr)],
                                  acc.at[pl.ds(zb, zr)], zsem).wait()
            plsc.subcore_barrier()

            @pl.when(nb > 0)
            def _():
                g_issue(0, 0)

            def quad(i4, carry):
                for b in range(NBUF):
                    j = i4 * NBUF + b

                    @pl.when(j < nb)
                    def _():
                        jn = j + 1
                        bn = (b + 1) % NBUF

                        @pl.when(jn < nb)
                        def _():
                            @pl.when(jn >= NBUF)
                            def _():
                                s_wait(bn)
                            g_issue(jn, bn)

                        g_wait(b)
                        s_issue(j, b)
                return carry

            lax.fori_loop(0, (nb + NBUF - 1) // NBUF, quad, 0)
            for b in range(NBUF):
                @pl.when(b < nb)
                def _():
                    s_wait(b)
            plsc.subcore_barrier()

            ob = lo + s * wr
            pltpu.sync_copy(acc.at[pl.ds(s * wr, wr)],
                            sums_o.at[pl.ds(ob, wr)])
            if p < P - 1:
                plsc.subcore_barrier()

    f = pl.kernel(
        body,
        out_type=[jax.ShapeDtypeStruct((n_out, HID), F32)],
        mesh=mesh, scratch_types=scratch,
        compiler_params=pltpu.CompilerParams(**_CPARAMS))
    _SEG_CACHE[key] = (f, et, n_out)
    return _SEG_CACHE[key]


_CNT_CACHE = {}


def _make_counts(configs):
    key = tuple(configs)
    if key in _CNT_CACHE:
        return _CNT_CACHE[key]

    geo = []
    for n_dst, n_edges in configs:
        et = _et_of(n_edges)
        chunk = _cdiv(n_dst, NC * 128) * 128
        geo.append((et, chunk))
    et_max = max(g[0] for g in geo)
    a_max = max(g[1] for g in geo) + 128
    CB = 128

    out_type = [jax.ShapeDtypeStruct((NC * g[1], 16), F32) for g in geo]
    scratch = [
        pltpu.VMEM((et_max,), I32),
        pltpu.VMEM((et_max + 16,), I32),
        pltpu.VMEM((CB, 16), F32),
        pltpu.VMEM((CB, 16), F32),
        pltpu.VMEM_SHARED((a_max, 16), F32),
    ]
    scratch += [pltpu.SemaphoreType.DMA] * (NBUF + 1)

    mesh = plsc.VectorSubcoreMesh(**_MESH)
    n_types = len(configs)

    def body(*refs):
        dst_hbms = refs[:n_types]
        outs = refs[n_types:2 * n_types]
        dst_raw, ldst, ones, zcnt, cnt = refs[2 * n_types:2 * n_types + 5]
        sems = refs[2 * n_types + 5:]
        csem = sems[:NBUF]
        zsem = sems[NBUF]

        c = lax.axis_index("c")
        s = lax.axis_index("s")

        zvec = jnp.zeros((16,), F32)
        ovec = jnp.ones((16,), F32)

        def init(r, carry):
            ones[r, :] = ovec
            zcnt[r, :] = zvec
            return carry

        lax.fori_loop(0, CB, init, 0)

        for t in range(n_types):
            et, chunk = geo[t]
            A = chunk + 128
            zr = A // 16
            wr = chunk // 16
            lo = c * chunk

            znf, zrem = divmod(zr, CB)
            zb = s * zr

            def z_descs():
                ds_ = []
                for q in range(znf):
                    ds_.append((zcnt, cnt.at[pl.ds(zb + q * CB, CB)]))
                if zrem:
                    ds_.append((zcnt.at[pl.ds(0, zrem)],
                                cnt.at[pl.ds(zb + znf * CB, zrem)]))
                return ds_

            for src_r, dst_r in z_descs():
                pltpu.async_copy(src_r, dst_r, zsem)

            base = s * et
            pltpu.sync_copy(dst_hbms[t].at[pl.ds(base, et)],
                            dst_raw.at[pl.ds(0, et)])

            tvec = jnp.full((16,), chunk, I32)

            def fill(i, carry):
                ldst[pl.ds(i * 16, 16)] = tvec
                return carry

            lax.fori_loop(0, et // 16 + 1, fill, 0)

            def scan(g, off):
                d = dst_raw[pl.ds(g * 16, 16)]
                m = (d >= lo) & (d < lo + chunk)
                plsc.store_compressed(ldst.at[pl.ds(off, 16)], d - lo, mask=m)
                return off + jnp.max(plsc.all_reduce_population_count(m))

            m_cnt = lax.fori_loop(0, et // 16, scan, jnp.int32(0))
            nb = (m_cnt + CB - 1) // CB

            for src_r, dst_r in z_descs():
                pltpu.make_async_copy(src_r, dst_r, zsem).wait()
            plsc.subcore_barrier()

            def c_issue(j, b):
                pltpu.async_copy(ones, cnt.at[ldst.at[pl.ds(j * CB, CB)]],
                                 csem[b], add=True)

            def c_wait(b):
                pltpu.make_async_copy(
                    ones, cnt.at[ldst.at[pl.ds(0, CB)]], csem[b]).wait()

            def quad(i4, carry):
                for b in range(NBUF):
                    j = i4 * NBUF + b

                    @pl.when(j < nb)
                    def _():
                        @pl.when(j >= NBUF)
                        def _():
                            c_wait(b)
                        c_issue(j, b)
                return carry

            lax.fori_loop(0, (nb + NBUF - 1) // NBUF, quad, 0)
            for b in range(NBUF):
                @pl.when(b < nb)
                def _():
                    c_wait(b)
            plsc.subcore_barrier()

            ob = lo + s * wr
            pltpu.sync_copy(cnt.at[pl.ds(s * wr, wr)],
                            outs[t].at[pl.ds(ob, wr)])
            if t < n_types - 1:
                plsc.subcore_barrier()

    f = pl.kernel(
        body, out_type=out_type, mesh=mesh, scratch_types=scratch,
        compiler_params=pltpu.CompilerParams(**_CPARAMS))
    _CNT_CACHE[key] = f
    return f


def _pad_edges(ei, n_edges_pad):
    e = ei.shape[1]
    pad = n_edges_pad - e
    src = jnp.concatenate([ei[0].astype(I32), jnp.zeros((pad,), I32)])
    dst = jnp.concatenate([ei[1].astype(I32), jnp.full((pad,), -1, I32)])
    return src, dst


def _mm_bias(x, w, b):
    n, kd = x.shape
    m = w.shape[1]
    grid = _cdiv(n, BR)

    def body(x_ref, w_ref, b_ref, o_ref):
        o_ref[...] = (
            jnp.dot(x_ref[...], w_ref[...], preferred_element_type=F32)
            + b_ref[...])

    return pl.pallas_call(
        body,
        grid=(grid,),
        in_specs=[
            pl.BlockSpec((BR, kd), lambda i: (i, 0)),
            pl.BlockSpec((kd, m), lambda i: (0, 0)),
            pl.BlockSpec((1, m), lambda i: (0, 0)),
        ],
        out_specs=pl.BlockSpec((BR, m), lambda i: (i, 0)),
        out_shape=jax.ShapeDtypeStruct((n, m), F32),
    )(x, w, b)


def _combine(h, sums, cnts, wl_stack, wr_sum, blm, g, b):
    n = h.shape[0]
    k = len(sums)
    grid = _cdiv(n, BR)

    def body(*refs):
        h_ref = refs[0]
        s_refs = refs[1:1 + k]
        c_refs = refs[1 + k:1 + 2 * k]
        wl_ref, wr_ref, blm_ref, g_ref, b_ref, o_ref = refs[1 + 2 * k:]
        hv = h_ref[...]
        acc = jnp.dot(hv, wr_ref[...], preferred_element_type=F32)
        for i in range(k):
            cntv = c_refs[i][...][:, 0:1]
            recip = 1.0 / jnp.maximum(cntv, 1.0)
            acc = acc + jnp.dot(s_refs[i][...] * recip, wl_ref[i],
                                preferred_element_type=F32)
        x = acc * (1.0 / k) + blm_ref[...]
        mu = jnp.mean(x, axis=-1, keepdims=True)
        var = jnp.mean((x - mu) ** 2, axis=-1, keepdims=True)
        xn = (x - mu) * lax.rsqrt(var + 1e-5) * g_ref[...] + b_ref[...]
        o_ref[...] = xn + hv

    in_specs = [pl.BlockSpec((BR, HID), lambda i: (i, 0))]
    in_specs += [pl.BlockSpec((BR, HID), lambda i: (i, 0))] * k
    in_specs += [pl.BlockSpec((BR, 16), lambda i: (i, 0))] * k
    in_specs += [
        pl.BlockSpec((k, HID, HID), lambda i: (0, 0, 0)),
        pl.BlockSpec((HID, HID), lambda i: (0, 0)),
        pl.BlockSpec((1, HID), lambda i: (0, 0)),
        pl.BlockSpec((1, HID), lambda i: (0, 0)),
        pl.BlockSpec((1, HID), lambda i: (0, 0)),
    ]
    return pl.pallas_call(
        body,
        grid=(grid,),
        in_specs=in_specs,
        out_specs=pl.BlockSpec((BR, HID), lambda i: (i, 0)),
        out_shape=jax.ShapeDtypeStruct((n, HID), F32),
    )(h, *sums, *cnts, wl_stack, wr_sum, blm, g, b)


def kernel(x_occ, x_chord, x_sec, ei_next, ei_prev, ei_inst, ei_inst_rev,
           ei_in_sec, ei_sec_rev, ei_next_sec, Wp_occ, bp_occ, Wp_chord,
           bp_chord, Wp_sec, bp_sec, Wl, bl, Wr, ln_g, ln_b, Wc, bc):
    n = {'occ': x_occ.shape[0], 'chord': x_chord.shape[0],
         'sec': x_sec.shape[0]}
    meta = [('occ', 'occ'), ('occ', 'occ'), ('occ', 'chord'),
            ('chord', 'occ'), ('occ', 'sec'), ('sec', 'occ'), ('sec', 'sec')]
    eis = [ei_next, ei_prev, ei_inst, ei_inst_rev, ei_in_sec, ei_sec_rev,
           ei_next_sec]
    incoming = {'occ': [0, 1, 3, 5], 'chord': [2], 'sec': [4, 6]}
    num_layers = Wl.shape[0]

    h = {'occ': _mm_bias(x_occ, Wp_occ, bp_occ[None]),
         'chord': _mm_bias(x_chord, Wp_chord, bp_chord[None]),
         'sec': _mm_bias(x_sec, Wp_sec, bp_sec[None])}

    seg = []
    eip = []
    for i, (st, dt) in enumerate(meta):
        f, et, n_out = _make_seg_sum(n[st], n[dt], eis[i].shape[1])
        seg.append(f)
        eip.append(_pad_edges(eis[i], NS * et))
    zeros_hbm = jnp.zeros((_A_MAX, HID), F32)

    cfg = tuple((n[dt], eis[i].shape[1]) for i, (st, dt) in enumerate(meta))
    fcnt = _make_counts(cfg)
    cnts = fcnt(*[eip[i][1] for i in range(len(meta))])
    cnts = list(cnts) if isinstance(cnts, (tuple, list)) else [cnts]

    for l in range(num_layers):
        sums = {}
        for i, (st, dt) in enumerate(meta):
            out = seg[i](h[st], eip[i][0], eip[i][1], zeros_hbm)
            sums[i] = out[0] if isinstance(out, (tuple, list)) else out
        h_new = {}
        for nt in ('occ', 'chord', 'sec'):
            idxs = incoming[nt]
            k = len(idxs)
            wl_stack = jnp.stack([Wl[l, i] for i in idxs])
            wr_sum = sum(Wr[l, i] for i in idxs)
            blm = (sum(bl[l, i] for i in idxs) / k)[None]
            h_new[nt] = _combine(
                h[nt], [sums[i] for i in idxs], [cnts[i] for i in idxs],
                wl_stack, wr_sum, blm, ln_g[l][None], ln_b[l][None])
        h = h_new

    return _mm_bias(h['occ'], Wc, bc[None])


# speedup vs baseline: 1.6433x; 1.6433x over previous
import jax
import jax.numpy as jnp
from jax import lax
from jax.experimental import pallas as pl
from jax.experimental.pallas import tpu as pltpu
from jax.experimental.pallas import tpu_sc as plsc

F32 = jnp.float32
I32 = jnp.int32
NC = 2
NS = 16
HID = 128
BR = 256
NBUF = 4
BLK = 64

_MESH = dict(core_axis_name="c", subcore_axis_name="s",
             num_cores=NC, num_subcores=NS)
_CPARAMS = dict(needs_layout_passes=False, use_tc_tiling_on_sc=False)


def _cdiv(a, b):
    return -(-a // b)


def _et_of(n_edges):
    return max(2, _cdiv(n_edges, NS * 128)) * 128


_SEG_CACHE = {}
_SPMEM_BUDGET = 4_700_000


def _seg_geometry(n_dst):
    p = 1
    while True:
        chunk = _cdiv(n_dst, NC * p * 128) * 128
        if (chunk + 128) * 512 <= _SPMEM_BUDGET:
            return p, chunk
        p += 1


_A_MAX = 8576


def _make_seg_sum(n_src, n_dst, n_edges):
    key = (n_src, n_dst, n_edges)
    if key in _SEG_CACHE:
        return _SEG_CACHE[key]

    et = _et_of(n_edges)
    P, chunk = _seg_geometry(n_dst)
    A = chunk + 128
    assert A <= _A_MAX
    n_out = NC * P * chunk
    zr = A // 16
    wr = chunk // 16

    scratch = [
        pltpu.VMEM((et,), I32),
        pltpu.VMEM((et,), I32),
        pltpu.VMEM((et + 16,), I32),
        pltpu.VMEM((et + 16,), I32),
        pltpu.VMEM((NBUF, BLK, HID), F32),
        pltpu.VMEM_SHARED((A, HID), F32),
    ]
    scratch += [pltpu.SemaphoreType.DMA] * (2 * NBUF + 1)

    mesh = plsc.VectorSubcoreMesh(**_MESH)

    def body(hsrc, src_hbm, dst_hbm, z_hbm, sums_o, src_raw, dst_raw,
             lsrc, ldst, rows, acc, *sems):
        gsem = sems[:NBUF]
        ssem = sems[NBUF:2 * NBUF]
        zsem = sems[2 * NBUF]

        c = lax.axis_index("c")
        s = lax.axis_index("s")

        base = s * et
        pltpu.sync_copy(src_hbm.at[pl.ds(base, et)], src_raw)
        pltpu.sync_copy(dst_hbm.at[pl.ds(base, et)], dst_raw)

        zb = s * zr

        def g_issue(j, b):
            pltpu.async_copy(
                hsrc.at[lsrc.at[pl.ds(j * BLK, BLK)]], rows.at[b], gsem[b])

        def g_wait(b):
            pltpu.make_async_copy(
                hsrc.at[lsrc.at[pl.ds(0, BLK)]], rows.at[b], gsem[b]).wait()

        def s_issue(j, b):
            pltpu.async_copy(rows.at[b],
                             acc.at[ldst.at[pl.ds(j * BLK, BLK)]],
                             ssem[b], add=True)

        def s_wait(b):
            pltpu.make_async_copy(
                rows.at[b], acc.at[ldst.at[pl.ds(0, BLK)]], ssem[b]).wait()

        for p in range(P):
            ri = c * P + p
            lo = ri * chunk

            pltpu.async_copy(z_hbm.at[pl.ds(0, zr)], acc.at[pl.ds(zb, zr)],
                             zsem)

            zivec = jnp.zeros((16,), I32)
            tvec = jnp.full((16,), chunk, I32)

            def fill(i, carry):
                lsrc[pl.ds(i * 16, 16)] = zivec
                ldst[pl.ds(i * 16, 16)] = tvec
                return carry

            lax.fori_loop(0, et // 16 + 1, fill, 0)

            def scan(g, off):
                d = dst_raw[pl.ds(g * 16, 16)]
                sv = src_raw[pl.ds(g * 16, 16)]
                m = (d >= lo) & (d < lo + chunk)
                plsc.store_compressed(lsrc.at[pl.ds(off, 16)], sv, mask=m)
                plsc.store_compressed(ldst.at[pl.ds(off, 16)], d - lo, mask=m)
                return off + jnp.max(plsc.all_reduce_population_count(m))

            m_cnt = lax.fori_loop(0, et // 16, scan, jnp.int32(0))
            nb = (m_cnt + BLK - 1) // BLK

            pltpu.make_async_copy(z_hbm.at[pl.ds(0, zr)],
                                  acc.at[pl.ds(zb, zr)], zsem).wait()
            plsc.subcore_barrier()

            @pl.when(nb > 0)
            def _():
                g_issue(0, 0)

            def quad(i4, carry):
                for b in range(NBUF):
                    j = i4 * NBUF + b

                    @pl.when(j < nb)
                    def _():
                        jn = j + 1
                        bn = (b + 1) % NBUF

                        @pl.when(jn < nb)
                        def _():
                            @pl.when(jn >= NBUF)
                            def _():
                                s_wait(bn)
                            g_issue(jn, bn)

                        g_wait(b)
                        s_issue(j, b)
                return carry

            lax.fori_loop(0, (nb + NBUF - 1) // NBUF, quad, 0)
            for b in range(NBUF):
                @pl.when(b < nb)
                def _():
                    s_wait(b)
            plsc.subcore_barrier()

            ob = lo + s * wr
            pltpu.sync_copy(acc.at[pl.ds(s * wr, wr)],
                            sums_o.at[pl.ds(ob, wr)])
            if p < P - 1:
                plsc.subcore_barrier()

    f = pl.kernel(
        body,
        out_type=[jax.ShapeDtypeStruct((n_out, HID), F32)],
        mesh=mesh, scratch_types=scratch,
        compiler_params=pltpu.CompilerParams(**_CPARAMS))
    _SEG_CACHE[key] = (f, et, n_out)
    return _SEG_CACHE[key]


_CNT_CACHE = {}


def _make_counts(configs):
    key = tuple(configs)
    if key in _CNT_CACHE:
        return _CNT_CACHE[key]

    geo = []
    for n_dst, n_edges in configs:
        et = _et_of(n_edges)
        chunk = _cdiv(n_dst, NC * 128) * 128
        geo.append((et, chunk))
    et_max = max(g[0] for g in geo)
    a_max = max(g[1] for g in geo) + 128
    CB = 128

    out_type = [jax.ShapeDtypeStruct((NC * g[1], 16), F32) for g in geo]
    scratch = [
        pltpu.VMEM((et_max,), I32),
        pltpu.VMEM((et_max + 16,), I32),
        pltpu.VMEM((CB, 16), F32),
        pltpu.VMEM((CB, 16), F32),
        pltpu.VMEM_SHARED((a_max, 16), F32),
    ]
    scratch += [pltpu.SemaphoreType.DMA] * (NBUF + 1)

    mesh = plsc.VectorSubcoreMesh(**_MESH)
    n_types = len(configs)

    def body(*refs):
        dst_hbms = refs[:n_types]
        outs = refs[n_types:2 * n_types]
        dst_raw, ldst, ones, zcnt, cnt = refs[2 * n_types:2 * n_types + 5]
        sems = refs[2 * n_types + 5:]
        csem = sems[:NBUF]
        zsem = sems[NBUF]

        c = lax.axis_index("c")
        s = lax.axis_index("s")

        zvec = jnp.zeros((16,), F32)
        ovec = jnp.ones((16,), F32)

        def init(r, carry):
            ones[r, :] = ovec
            zcnt[r, :] = zvec
            return carry

        lax.fori_loop(0, CB, init, 0)

        for t in range(n_types):
            et, chunk = geo[t]
            A = chunk + 128
            zr = A // 16
            wr = chunk // 16
            lo = c * chunk

            znf, zrem = divmod(zr, CB)
            zb = s * zr

            def z_descs():
                ds_ = []
                for q in range(znf):
                    ds_.append((zcnt, cnt.at[pl.ds(zb + q * CB, CB)]))
                if zrem:
                    ds_.append((zcnt.at[pl.ds(0, zrem)],
                                cnt.at[pl.ds(zb + znf * CB, zrem)]))
                return ds_

            for src_r, dst_r in z_descs():
                pltpu.async_copy(src_r, dst_r, zsem)

            base = s * et
            pltpu.sync_copy(dst_hbms[t].at[pl.ds(base, et)],
                            dst_raw.at[pl.ds(0, et)])

            tvec = jnp.full((16,), chunk, I32)

            def fill(i, carry):
                ldst[pl.ds(i * 16, 16)] = tvec
                return carry

            lax.fori_loop(0, et // 16 + 1, fill, 0)

            def scan(g, off):
                d = dst_raw[pl.ds(g * 16, 16)]
                m = (d >= lo) & (d < lo + chunk)
                plsc.store_compressed(ldst.at[pl.ds(off, 16)], d - lo, mask=m)
                return off + jnp.max(plsc.all_reduce_population_count(m))

            m_cnt = lax.fori_loop(0, et // 16, scan, jnp.int32(0))
            nb = (m_cnt + CB - 1) // CB

            for src_r, dst_r in z_descs():
                pltpu.make_async_copy(src_r, dst_r, zsem).wait()
            plsc.subcore_barrier()

            def c_issue(j, b):
                pltpu.async_copy(ones, cnt.at[ldst.at[pl.ds(j * CB, CB)]],
                                 csem[b], add=True)

            def c_wait(b):
                pltpu.make_async_copy(
                    ones, cnt.at[ldst.at[pl.ds(0, CB)]], csem[b]).wait()

            def quad(i4, carry):
                for b in range(NBUF):
                    j = i4 * NBUF + b

                    @pl.when(j < nb)
                    def _():
                        @pl.when(j >= NBUF)
                        def _():
                            c_wait(b)
                        c_issue(j, b)
                return carry

            lax.fori_loop(0, (nb + NBUF - 1) // NBUF, quad, 0)
            for b in range(NBUF):
                @pl.when(b < nb)
                def _():
                    c_wait(b)
            plsc.subcore_barrier()

            ob = lo + s * wr
            pltpu.sync_copy(cnt.at[pl.ds(s * wr, wr)],
                            outs[t].at[pl.ds(ob, wr)])
            if t < n_types - 1:
                plsc.subcore_barrier()

    f = pl.kernel(
        body, out_type=out_type, mesh=mesh, scratch_types=scratch,
        compiler_params=pltpu.CompilerParams(**_CPARAMS))
    _CNT_CACHE[key] = f
    return f


def _pad_edges(ei, n_edges_pad):
    e = ei.shape[1]
    pad = n_edges_pad - e
    src = jnp.concatenate([ei[0].astype(I32), jnp.zeros((pad,), I32)])
    dst = jnp.concatenate([ei[1].astype(I32), jnp.full((pad,), -1, I32)])
    return src, dst


def _mm_bias(x, w, b):
    n, kd = x.shape
    m = w.shape[1]
    grid = _cdiv(n, BR)

    def body(x_ref, w_ref, b_ref, o_ref):
        o_ref[...] = (
            jnp.dot(x_ref[...], w_ref[...], preferred_element_type=F32)
            + b_ref[...])

    return pl.pallas_call(
        body,
        grid=(grid,),
        in_specs=[
            pl.BlockSpec((BR, kd), lambda i: (i, 0)),
            pl.BlockSpec((kd, m), lambda i: (0, 0)),
            pl.BlockSpec((1, m), lambda i: (0, 0)),
        ],
        out_specs=pl.BlockSpec((BR, m), lambda i: (i, 0)),
        out_shape=jax.ShapeDtypeStruct((n, m), F32),
    )(x, w, b)


def _combine(h, sums, cnts, wl_stack, wr_sum, blm, g, b):
    n = h.shape[0]
    k = len(sums)
    grid = _cdiv(n, BR)

    def body(*refs):
        h_ref = refs[0]
        s_refs = refs[1:1 + k]
        c_refs = refs[1 + k:1 + 2 * k]
        wl_ref, wr_ref, blm_ref, g_ref, b_ref, o_ref = refs[1 + 2 * k:]
        hv = h_ref[...]
        acc = jnp.dot(hv, wr_ref[...], preferred_element_type=F32)
        for i in range(k):
            cntv = c_refs[i][...][:, 0:1]
            recip = 1.0 / jnp.maximum(cntv, 1.0)
            acc = acc + jnp.dot(s_refs[i][...] * recip, wl_ref[i],
                                preferred_element_type=F32)
        x = acc * (1.0 / k) + blm_ref[...]
        mu = jnp.mean(x, axis=-1, keepdims=True)
        var = jnp.mean((x - mu) ** 2, axis=-1, keepdims=True)
        xn = (x - mu) * lax.rsqrt(var + 1e-5) * g_ref[...] + b_ref[...]
        o_ref[...] = xn + hv

    in_specs = [pl.BlockSpec((BR, HID), lambda i: (i, 0))]
    in_specs += [pl.BlockSpec((BR, HID), lambda i: (i, 0))] * k
    in_specs += [pl.BlockSpec((BR, 16), lambda i: (i, 0))] * k
    in_specs += [
        pl.BlockSpec((k, HID, HID), lambda i: (0, 0, 0)),
        pl.BlockSpec((HID, HID), lambda i: (0, 0)),
        pl.BlockSpec((1, HID), lambda i: (0, 0)),
        pl.BlockSpec((1, HID), lambda i: (0, 0)),
        pl.BlockSpec((1, HID), lambda i: (0, 0)),
    ]
    return pl.pallas_call(
        body,
        grid=(grid,),
        in_specs=in_specs,
        out_specs=pl.BlockSpec((BR, HID), lambda i: (i, 0)),
        out_shape=jax.ShapeDtypeStruct((n, HID), F32),
    )(h, *sums, *cnts, wl_stack, wr_sum, blm, g, b)


def kernel(x_occ, x_chord, x_sec, ei_next, ei_prev, ei_inst, ei_inst_rev,
           ei_in_sec, ei_sec_rev, ei_next_sec, Wp_occ, bp_occ, Wp_chord,
           bp_chord, Wp_sec, bp_sec, Wl, bl, Wr, ln_g, ln_b, Wc, bc):
    n = {'occ': x_occ.shape[0], 'chord': x_chord.shape[0],
         'sec': x_sec.shape[0]}
    meta = [('occ', 'occ'), ('occ', 'occ'), ('occ', 'chord'),
            ('chord', 'occ'), ('occ', 'sec'), ('sec', 'occ'), ('sec', 'sec')]
    eis = [ei_next, ei_prev, ei_inst, ei_inst_rev, ei_in_sec, ei_sec_rev,
           ei_next_sec]
    incoming = {'occ': [0, 1, 3, 5], 'chord': [2], 'sec': [4, 6]}
    num_layers = Wl.shape[0]

    h = {'occ': _mm_bias(x_occ, Wp_occ, bp_occ[None]),
         'chord': _mm_bias(x_chord, Wp_chord, bp_chord[None]),
         'sec': _mm_bias(x_sec, Wp_sec, bp_sec[None])}

    seg = []
    eip = []
    for i, (st, dt) in enumerate(meta):
        f, et, n_out = _make_seg_sum(n[st], n[dt], eis[i].shape[1])
        seg.append(f)
        eip.append(_pad_edges(eis[i], NS * et))
    zeros_hbm = jnp.zeros((_A_MAX, HID), F32)

    cfg = tuple((n[dt], eis[i].shape[1]) for i, (st, dt) in enumerate(meta))
    fcnt = _make_counts(cfg)
    cnts = fcnt(*[eip[i][1] for i in range(len(meta))])
    cnts = list(cnts) if isinstance(cnts, (tuple, list)) else [cnts]

    for l in range(num_layers):
        sums = {}
        for i, (st, dt) in enumerate(meta):
            out = seg[i](h[st], eip[i][0], eip[i][1], zeros_hbm)
            sums[i] = out[0] if isinstance(out, (tuple, list)) else out
        h_new = {}
        for nt in ('occ', 'chord', 'sec'):
            idxs = incoming[nt]
            k = len(idxs)
            wl_stack = jnp.stack([Wl[l, i] for i in idxs])
            wr_sum = sum(Wr[l, i] for i in idxs)
            blm = (sum(bl[l, i] for i in idxs) / k)[None]
            h_new[nt] = _combine(
                h[nt], [sums[i] for i in idxs], [cnts[i] for i in idxs],
                wl_stack, wr_sum, blm, ln_g[l][None], ln_b[l][None])
        h = h_new

    return _mm_bias(h['occ'], Wc, bc[None])


# speedup vs baseline: 1.7157x; 1.0441x over previous
import jax
import jax.numpy as jnp
from jax import lax
from jax.experimental import pallas as pl
from jax.experimental.pallas import tpu as pltpu
from jax.experimental.pallas import tpu_sc as plsc

F32 = jnp.float32
I32 = jnp.int32
NC = 2
NS = 16
HID = 128
BR = 256
NBUF = 4
BLK = 64

_MESH = dict(core_axis_name="c", subcore_axis_name="s",
             num_cores=NC, num_subcores=NS)
_CPARAMS = dict(needs_layout_passes=False, use_tc_tiling_on_sc=False)


def _cdiv(a, b):
    return -(-a // b)


def _et_of(n_edges):
    return max(2, _cdiv(n_edges, NS * 128)) * 128


_SEG_CACHE = {}
_SPMEM_BUDGET = 4_700_000


def _seg_geometry(n_dst):
    p = 1
    while True:
        chunk = _cdiv(n_dst, NC * p * 128) * 128
        if (chunk + 128) * 512 <= _SPMEM_BUDGET:
            return p, chunk
        p += 1


_A_MAX = 8576


def _make_seg_sum(n_src, n_dst, n_edges):
    key = (n_src, n_dst, n_edges)
    if key in _SEG_CACHE:
        return _SEG_CACHE[key]

    et = _et_of(n_edges)
    P, chunk = _seg_geometry(n_dst)
    A = chunk + 128
    assert A <= _A_MAX
    n_out = NC * P * chunk
    zr = A // 16
    wr = chunk // 16

    scratch = [
        pltpu.VMEM((et,), I32),
        pltpu.VMEM((et,), I32),
        pltpu.VMEM((et + BLK,), I32),
        pltpu.VMEM((et + BLK,), I32),
        pltpu.VMEM((NBUF, BLK, HID), F32),
        pltpu.VMEM((64, HID), F32),
        pltpu.VMEM_SHARED((A, HID), F32),
    ]
    scratch += [pltpu.SemaphoreType.DMA] * (2 * NBUF + 1)

    mesh = plsc.VectorSubcoreMesh(**_MESH)

    def body(hsrc, src_hbm, dst_hbm, sums_o, src_raw, dst_raw,
             lsrc, ldst, rows, zrow, acc, *sems):
        gsem = sems[:NBUF]
        ssem = sems[NBUF:2 * NBUF]
        zsem = sems[2 * NBUF]

        c = lax.axis_index("c")
        s = lax.axis_index("s")

        zvec = jnp.zeros((16,), F32)

        def init(r, carry):
            for v in range(HID // 16):
                zrow[r, pl.ds(v * 16, 16)] = zvec
            return carry

        lax.fori_loop(0, 64, init, 0)

        base = s * et
        pltpu.sync_copy(src_hbm.at[pl.ds(base, et)], src_raw)
        pltpu.sync_copy(dst_hbm.at[pl.ds(base, et)], dst_raw)

        zb = s * zr
        znf, zrem = divmod(zr, 64)

        def z_descs():
            ds_ = []
            for q in range(znf):
                ds_.append((zrow, acc.at[pl.ds(zb + q * 64, 64)]))
            if zrem:
                ds_.append((zrow.at[pl.ds(0, zrem)],
                            acc.at[pl.ds(zb + znf * 64, zrem)]))
            return ds_

        def g_issue(j, b):
            pltpu.async_copy(
                hsrc.at[lsrc.at[pl.ds(j * BLK, BLK)]], rows.at[b], gsem[b])

        def g_wait(b):
            pltpu.make_async_copy(
                hsrc.at[lsrc.at[pl.ds(0, BLK)]], rows.at[b], gsem[b]).wait()

        def s_issue(j, b):
            pltpu.async_copy(rows.at[b],
                             acc.at[ldst.at[pl.ds(j * BLK, BLK)]],
                             ssem[b], add=True)

        def s_wait(b):
            pltpu.make_async_copy(
                rows.at[b], acc.at[ldst.at[pl.ds(0, BLK)]], ssem[b]).wait()

        for p in range(P):
            ri = c * P + p
            lo = ri * chunk

            for src_r, dst_r in z_descs():
                pltpu.async_copy(src_r, dst_r, zsem)

            zivec = jnp.zeros((16,), I32)
            tvec = jnp.full((16,), chunk, I32)

            def scan(g, off):
                d = dst_raw[pl.ds(g * 16, 16)]
                sv = src_raw[pl.ds(g * 16, 16)]
                m = (d >= lo) & (d < lo + chunk)
                plsc.store_compressed(lsrc.at[pl.ds(off, 16)], sv, mask=m)
                plsc.store_compressed(ldst.at[pl.ds(off, 16)], d - lo, mask=m)
                return off + jnp.max(plsc.all_reduce_population_count(m))

            m_cnt = lax.fori_loop(0, et // 16, scan, jnp.int32(0))
            nb = (m_cnt + BLK - 1) // BLK

            for g in range(BLK // 16):
                lsrc[pl.ds(m_cnt + g * 16, 16)] = zivec
                ldst[pl.ds(m_cnt + g * 16, 16)] = tvec

            for src_r, dst_r in z_descs():
                pltpu.make_async_copy(src_r, dst_r, zsem).wait()
            plsc.subcore_barrier()

            @pl.when(nb > 0)
            def _():
                g_issue(0, 0)

            def quad(i4, carry):
                for b in range(NBUF):
                    j = i4 * NBUF + b

                    @pl.when(j < nb)
                    def _():
                        jn = j + 1
                        bn = (b + 1) % NBUF

                        @pl.when(jn < nb)
                        def _():
                            @pl.when(jn >= NBUF)
                            def _():
                                s_wait(bn)
                            g_issue(jn, bn)

                        g_wait(b)
                        s_issue(j, b)
                return carry

            lax.fori_loop(0, (nb + NBUF - 1) // NBUF, quad, 0)
            for b in range(NBUF):
                @pl.when(b < nb)
                def _():
                    s_wait(b)
            plsc.subcore_barrier()

            ob = lo + s * wr
            pltpu.sync_copy(acc.at[pl.ds(s * wr, wr)],
                            sums_o.at[pl.ds(ob, wr)])
            if p < P - 1:
                plsc.subcore_barrier()

    f = pl.kernel(
        body,
        out_type=[jax.ShapeDtypeStruct((n_out, HID), F32)],
        mesh=mesh, scratch_types=scratch,
        compiler_params=pltpu.CompilerParams(**_CPARAMS))
    _SEG_CACHE[key] = (f, et, n_out)
    return _SEG_CACHE[key]


_CNT_CACHE = {}


def _make_counts(configs):
    key = tuple(configs)
    if key in _CNT_CACHE:
        return _CNT_CACHE[key]

    geo = []
    for n_dst, n_edges in configs:
        et = _et_of(n_edges)
        chunk = _cdiv(n_dst, NC * 128) * 128
        geo.append((et, chunk))
    et_max = max(g[0] for g in geo)
    a_max = max(g[1] for g in geo) + 128
    CB = 128

    out_type = [jax.ShapeDtypeStruct((NC * g[1], 16), F32) for g in geo]
    scratch = [
        pltpu.VMEM((et_max,), I32),
        pltpu.VMEM((et_max + 16,), I32),
        pltpu.VMEM((CB, 16), F32),
        pltpu.VMEM((CB, 16), F32),
        pltpu.VMEM_SHARED((a_max, 16), F32),
    ]
    scratch += [pltpu.SemaphoreType.DMA] * (NBUF + 1)

    mesh = plsc.VectorSubcoreMesh(**_MESH)
    n_types = len(configs)

    def body(*refs):
        dst_hbms = refs[:n_types]
        outs = refs[n_types:2 * n_types]
        dst_raw, ldst, ones, zcnt, cnt = refs[2 * n_types:2 * n_types + 5]
        sems = refs[2 * n_types + 5:]
        csem = sems[:NBUF]
        zsem = sems[NBUF]

        c = lax.axis_index("c")
        s = lax.axis_index("s")

        zvec = jnp.zeros((16,), F32)
        ovec = jnp.ones((16,), F32)

        def init(r, carry):
            ones[r, :] = ovec
            zcnt[r, :] = zvec
            return carry

        lax.fori_loop(0, CB, init, 0)

        for t in range(n_types):
            et, chunk = geo[t]
            A = chunk + 128
            zr = A // 16
            wr = chunk // 16
            lo = c * chunk

            znf, zrem = divmod(zr, CB)
            zb = s * zr

            def z_descs():
                ds_ = []
                for q in range(znf):
                    ds_.append((zcnt, cnt.at[pl.ds(zb + q * CB, CB)]))
                if zrem:
                    ds_.append((zcnt.at[pl.ds(0, zrem)],
                                cnt.at[pl.ds(zb + znf * CB, zrem)]))
                return ds_

            for src_r, dst_r in z_descs():
                pltpu.async_copy(src_r, dst_r, zsem)

            base = s * et
            pltpu.sync_copy(dst_hbms[t].at[pl.ds(base, et)],
                            dst_raw.at[pl.ds(0, et)])

            tvec = jnp.full((16,), chunk, I32)

            def fill(i, carry):
                ldst[pl.ds(i * 16, 16)] = tvec
                return carry

            lax.fori_loop(0, et // 16 + 1, fill, 0)

            def scan(g, off):
                d = dst_raw[pl.ds(g * 16, 16)]
                m = (d >= lo) & (d < lo + chunk)
                plsc.store_compressed(ldst.at[pl.ds(off, 16)], d - lo, mask=m)
                return off + jnp.max(plsc.all_reduce_population_count(m))

            m_cnt = lax.fori_loop(0, et // 16, scan, jnp.int32(0))
            nb = (m_cnt + CB - 1) // CB

            for src_r, dst_r in z_descs():
                pltpu.make_async_copy(src_r, dst_r, zsem).wait()
            plsc.subcore_barrier()

            def c_issue(j, b):
                pltpu.async_copy(ones, cnt.at[ldst.at[pl.ds(j * CB, CB)]],
                                 csem[b], add=True)

            def c_wait(b):
                pltpu.make_async_copy(
                    ones, cnt.at[ldst.at[pl.ds(0, CB)]], csem[b]).wait()

            def quad(i4, carry):
                for b in range(NBUF):
                    j = i4 * NBUF + b

                    @pl.when(j < nb)
                    def _():
                        @pl.when(j >= NBUF)
                        def _():
                            c_wait(b)
                        c_issue(j, b)
                return carry

            lax.fori_loop(0, (nb + NBUF - 1) // NBUF, quad, 0)
            for b in range(NBUF):
                @pl.when(b < nb)
                def _():
                    c_wait(b)
            plsc.subcore_barrier()

            ob = lo + s * wr
            pltpu.sync_copy(cnt.at[pl.ds(s * wr, wr)],
                            outs[t].at[pl.ds(ob, wr)])
            if t < n_types - 1:
                plsc.subcore_barrier()

    f = pl.kernel(
        body, out_type=out_type, mesh=mesh, scratch_types=scratch,
        compiler_params=pltpu.CompilerParams(**_CPARAMS))
    _CNT_CACHE[key] = f
    return f


def _pad_edges(ei, n_edges_pad):
    e = ei.shape[1]
    pad = n_edges_pad - e
    src = jnp.concatenate([ei[0].astype(I32), jnp.zeros((pad,), I32)])
    dst = jnp.concatenate([ei[1].astype(I32), jnp.full((pad,), -1, I32)])
    return src, dst


def _mm_bias(x, w, b):
    n, kd = x.shape
    m = w.shape[1]
    grid = _cdiv(n, BR)

    def body(x_ref, w_ref, b_ref, o_ref):
        o_ref[...] = (
            jnp.dot(x_ref[...], w_ref[...], preferred_element_type=F32)
            + b_ref[...])

    return pl.pallas_call(
        body,
        grid=(grid,),
        in_specs=[
            pl.BlockSpec((BR, kd), lambda i: (i, 0)),
            pl.BlockSpec((kd, m), lambda i: (0, 0)),
            pl.BlockSpec((1, m), lambda i: (0, 0)),
        ],
        out_specs=pl.BlockSpec((BR, m), lambda i: (i, 0)),
        out_shape=jax.ShapeDtypeStruct((n, m), F32),
    )(x, w, b)


def _combine(h, sums, cnts, wl_stack, wr_sum, blm, g, b):
    n = h.shape[0]
    k = len(sums)
    grid = _cdiv(n, BR)

    def body(*refs):
        h_ref = refs[0]
        s_refs = refs[1:1 + k]
        c_refs = refs[1 + k:1 + 2 * k]
        wl_ref, wr_ref, blm_ref, g_ref, b_ref, o_ref = refs[1 + 2 * k:]
        hv = h_ref[...]
        acc = jnp.dot(hv, wr_ref[...], preferred_element_type=F32)
        for i in range(k):
            cntv = c_refs[i][...][:, 0:1]
            recip = 1.0 / jnp.maximum(cntv, 1.0)
            acc = acc + jnp.dot(s_refs[i][...] * recip, wl_ref[i],
                                preferred_element_type=F32)
        x = acc * (1.0 / k) + blm_ref[...]
        mu = jnp.mean(x, axis=-1, keepdims=True)
        var = jnp.mean((x - mu) ** 2, axis=-1, keepdims=True)
        xn = (x - mu) * lax.rsqrt(var + 1e-5) * g_ref[...] + b_ref[...]
        o_ref[...] = xn + hv

    in_specs = [pl.BlockSpec((BR, HID), lambda i: (i, 0))]
    in_specs += [pl.BlockSpec((BR, HID), lambda i: (i, 0))] * k
    in_specs += [pl.BlockSpec((BR, 16), lambda i: (i, 0))] * k
    in_specs += [
        pl.BlockSpec((k, HID, HID), lambda i: (0, 0, 0)),
        pl.BlockSpec((HID, HID), lambda i: (0, 0)),
        pl.BlockSpec((1, HID), lambda i: (0, 0)),
        pl.BlockSpec((1, HID), lambda i: (0, 0)),
        pl.BlockSpec((1, HID), lambda i: (0, 0)),
    ]
    return pl.pallas_call(
        body,
        grid=(grid,),
        in_specs=in_specs,
        out_specs=pl.BlockSpec((BR, HID), lambda i: (i, 0)),
        out_shape=jax.ShapeDtypeStruct((n, HID), F32),
    )(h, *sums, *cnts, wl_stack, wr_sum, blm, g, b)


def kernel(x_occ, x_chord, x_sec, ei_next, ei_prev, ei_inst, ei_inst_rev,
           ei_in_sec, ei_sec_rev, ei_next_sec, Wp_occ, bp_occ, Wp_chord,
           bp_chord, Wp_sec, bp_sec, Wl, bl, Wr, ln_g, ln_b, Wc, bc):
    n = {'occ': x_occ.shape[0], 'chord': x_chord.shape[0],
         'sec': x_sec.shape[0]}
    meta = [('occ', 'occ'), ('occ', 'occ'), ('occ', 'chord'),
            ('chord', 'occ'), ('occ', 'sec'), ('sec', 'occ'), ('sec', 'sec')]
    eis = [ei_next, ei_prev, ei_inst, ei_inst_rev, ei_in_sec, ei_sec_rev,
           ei_next_sec]
    incoming = {'occ': [0, 1, 3, 5], 'chord': [2], 'sec': [4, 6]}
    num_layers = Wl.shape[0]

    h = {'occ': _mm_bias(x_occ, Wp_occ, bp_occ[None]),
         'chord': _mm_bias(x_chord, Wp_chord, bp_chord[None]),
         'sec': _mm_bias(x_sec, Wp_sec, bp_sec[None])}

    seg = []
    eip = []
    for i, (st, dt) in enumerate(meta):
        f, et, n_out = _make_seg_sum(n[st], n[dt], eis[i].shape[1])
        seg.append(f)
        eip.append(_pad_edges(eis[i], NS * et))

    cfg = tuple((n[dt], eis[i].shape[1]) for i, (st, dt) in enumerate(meta))
    fcnt = _make_counts(cfg)
    cnts = fcnt(*[eip[i][1] for i in range(len(meta))])
    cnts = list(cnts) if isinstance(cnts, (tuple, list)) else [cnts]

    for l in range(num_layers):
        sums = {}
        for i, (st, dt) in enumerate(meta):
            out = seg[i](h[st], eip[i][0], eip[i][1])
            sums[i] = out[0] if isinstance(out, (tuple, list)) else out
        h_new = {}
        for nt in ('occ', 'chord', 'sec'):
            idxs = incoming[nt]
            k = len(idxs)
            wl_stack = jnp.stack([Wl[l, i] for i in idxs])
            wr_sum = sum(Wr[l, i] for i in idxs)
            blm = (sum(bl[l, i] for i in idxs) / k)[None]
            h_new[nt] = _combine(
                h[nt], [sums[i] for i in idxs], [cnts[i] for i in idxs],
                wl_stack, wr_sum, blm, ln_g[l][None], ln_b[l][None])
        h = h_new

    return _mm_bias(h['occ'], Wc, bc[None])


# speedup vs baseline: 2.2098x; 1.2879x over previous
import jax
import jax.numpy as jnp
from jax import lax
from jax.experimental import pallas as pl
from jax.experimental.pallas import tpu as pltpu
from jax.experimental.pallas import tpu_sc as plsc

F32 = jnp.float32
I32 = jnp.int32
NC = 2
NS = 16
HID = 128
BR = 256
NBUF = 8
BLK = 32

_MESH = dict(core_axis_name="c", subcore_axis_name="s",
             num_cores=NC, num_subcores=NS)
_CPARAMS = dict(needs_layout_passes=False, use_tc_tiling_on_sc=False)


def _cdiv(a, b):
    return -(-a // b)


def _et_of(n_edges):
    return max(2, _cdiv(n_edges, NS * 128)) * 128


_SEG_CACHE = {}
_SPMEM_BUDGET = 4_700_000


def _seg_geometry(n_dst):
    p = 1
    while True:
        chunk = _cdiv(n_dst, NC * p * 128) * 128
        if (chunk + 128) * 512 <= _SPMEM_BUDGET:
            return p, chunk
        p += 1


_A_MAX = 8576


def _make_seg_sum(n_src, n_dst, n_edges):
    key = (n_src, n_dst, n_edges)
    if key in _SEG_CACHE:
        return _SEG_CACHE[key]

    et = _et_of(n_edges)
    P, chunk = _seg_geometry(n_dst)
    A = chunk + 128
    assert A <= _A_MAX
    n_out = NC * P * chunk
    zr = A // 16
    wr = chunk // 16

    scratch = [
        pltpu.VMEM((et,), I32),
        pltpu.VMEM((et,), I32),
        pltpu.VMEM((et + BLK,), I32),
        pltpu.VMEM((et + BLK,), I32),
        pltpu.VMEM((NBUF, BLK, HID), F32),
        pltpu.VMEM((64, HID), F32),
        pltpu.VMEM_SHARED((A, HID), F32),
    ]
    scratch += [pltpu.SemaphoreType.DMA] * (2 * NBUF + 1)

    mesh = plsc.VectorSubcoreMesh(**_MESH)

    def body(hsrc, src_hbm, dst_hbm, sums_o, src_raw, dst_raw,
             lsrc, ldst, rows, zrow, acc, *sems):
        gsem = sems[:NBUF]
        ssem = sems[NBUF:2 * NBUF]
        zsem = sems[2 * NBUF]

        c = lax.axis_index("c")
        s = lax.axis_index("s")

        zvec = jnp.zeros((16,), F32)

        def init(r, carry):
            for v in range(HID // 16):
                zrow[r, pl.ds(v * 16, 16)] = zvec
            return carry

        lax.fori_loop(0, 64, init, 0)

        base = s * et
        pltpu.sync_copy(src_hbm.at[pl.ds(base, et)], src_raw)
        pltpu.sync_copy(dst_hbm.at[pl.ds(base, et)], dst_raw)

        zb = s * zr
        znf, zrem = divmod(zr, 64)

        def z_descs():
            ds_ = []
            for q in range(znf):
                ds_.append((zrow, acc.at[pl.ds(zb + q * 64, 64)]))
            if zrem:
                ds_.append((zrow.at[pl.ds(0, zrem)],
                            acc.at[pl.ds(zb + znf * 64, zrem)]))
            return ds_

        def g_issue(j, b):
            pltpu.async_copy(
                hsrc.at[lsrc.at[pl.ds(j * BLK, BLK)]], rows.at[b], gsem[b])

        def g_wait(b):
            pltpu.make_async_copy(
                hsrc.at[lsrc.at[pl.ds(0, BLK)]], rows.at[b], gsem[b]).wait()

        def s_issue(j, b):
            pltpu.async_copy(rows.at[b],
                             acc.at[ldst.at[pl.ds(j * BLK, BLK)]],
                             ssem[b], add=True)

        def s_wait(b):
            pltpu.make_async_copy(
                rows.at[b], acc.at[ldst.at[pl.ds(0, BLK)]], ssem[b]).wait()

        for p in range(P):
            ri = c * P + p
            lo = ri * chunk

            for src_r, dst_r in z_descs():
                pltpu.async_copy(src_r, dst_r, zsem)

            zivec = jnp.zeros((16,), I32)
            tvec = jnp.full((16,), chunk, I32)

            def scan(g, off):
                d = dst_raw[pl.ds(g * 16, 16)]
                sv = src_raw[pl.ds(g * 16, 16)]
                m = (d >= lo) & (d < lo + chunk)
                plsc.store_compressed(lsrc.at[pl.ds(off, 16)], sv, mask=m)
                plsc.store_compressed(ldst.at[pl.ds(off, 16)], d - lo, mask=m)
                return off + jnp.max(plsc.all_reduce_population_count(m))

            m_cnt = lax.fori_loop(0, et // 16, scan, jnp.int32(0))
            nb = (m_cnt + BLK - 1) // BLK

            for g in range(BLK // 16):
                lsrc[pl.ds(m_cnt + g * 16, 16)] = zivec
                ldst[pl.ds(m_cnt + g * 16, 16)] = tvec

            for src_r, dst_r in z_descs():
                pltpu.make_async_copy(src_r, dst_r, zsem).wait()
            plsc.subcore_barrier()

            @pl.when(nb > 0)
            def _():
                g_issue(0, 0)

            def quad(i4, carry):
                for b in range(NBUF):
                    j = i4 * NBUF + b

                    @pl.when(j < nb)
                    def _():
                        jn = j + 1
                        bn = (b + 1) % NBUF

                        @pl.when(jn < nb)
                        def _():
                            @pl.when(jn >= NBUF)
                            def _():
                                s_wait(bn)
                            g_issue(jn, bn)

                        g_wait(b)
                        s_issue(j, b)
                return carry

            lax.fori_loop(0, (nb + NBUF - 1) // NBUF, quad, 0)
            for b in range(NBUF):
                @pl.when(b < nb)
                def _():
                    s_wait(b)
            plsc.subcore_barrier()

            ob = lo + s * wr
            pltpu.sync_copy(acc.at[pl.ds(s * wr, wr)],
                            sums_o.at[pl.ds(ob, wr)])
            if p < P - 1:
                plsc.subcore_barrier()

    f = pl.kernel(
        body,
        out_type=[jax.ShapeDtypeStruct((n_out, HID), F32)],
        mesh=mesh, scratch_types=scratch,
        compiler_params=pltpu.CompilerParams(**_CPARAMS))
    _SEG_CACHE[key] = (f, et, n_out)
    return _SEG_CACHE[key]


_CNT_CACHE = {}


def _make_counts(configs):
    key = tuple(configs)
    if key in _CNT_CACHE:
        return _CNT_CACHE[key]

    geo = []
    for n_dst, n_edges in configs:
        et = _et_of(n_edges)
        chunk = _cdiv(n_dst, NC * 128) * 128
        geo.append((et, chunk))
    et_max = max(g[0] for g in geo)
    a_max = max(g[1] for g in geo) + 128
    CB = 128

    out_type = [jax.ShapeDtypeStruct((NC * g[1], 16), F32) for g in geo]
    scratch = [
        pltpu.VMEM((et_max,), I32),
        pltpu.VMEM((et_max + 16,), I32),
        pltpu.VMEM((CB, 16), F32),
        pltpu.VMEM((CB, 16), F32),
        pltpu.VMEM_SHARED((a_max, 16), F32),
    ]
    scratch += [pltpu.SemaphoreType.DMA] * (NBUF + 1)

    mesh = plsc.VectorSubcoreMesh(**_MESH)
    n_types = len(configs)

    def body(*refs):
        dst_hbms = refs[:n_types]
        outs = refs[n_types:2 * n_types]
        dst_raw, ldst, ones, zcnt, cnt = refs[2 * n_types:2 * n_types + 5]
        sems = refs[2 * n_types + 5:]
        csem = sems[:NBUF]
        zsem = sems[NBUF]

        c = lax.axis_index("c")
        s = lax.axis_index("s")

        zvec = jnp.zeros((16,), F32)
        ovec = jnp.ones((16,), F32)

        def init(r, carry):
            ones[r, :] = ovec
            zcnt[r, :] = zvec
            return carry

        lax.fori_loop(0, CB, init, 0)

        for t in range(n_types):
            et, chunk = geo[t]
            A = chunk + 128
            zr = A // 16
            wr = chunk // 16
            lo = c * chunk

            znf, zrem = divmod(zr, CB)
            zb = s * zr

            def z_descs():
                ds_ = []
                for q in range(znf):
                    ds_.append((zcnt, cnt.at[pl.ds(zb + q * CB, CB)]))
                if zrem:
                    ds_.append((zcnt.at[pl.ds(0, zrem)],
                                cnt.at[pl.ds(zb + znf * CB, zrem)]))
                return ds_

            for src_r, dst_r in z_descs():
                pltpu.async_copy(src_r, dst_r, zsem)

            base = s * et
            pltpu.sync_copy(dst_hbms[t].at[pl.ds(base, et)],
                            dst_raw.at[pl.ds(0, et)])

            tvec = jnp.full((16,), chunk, I32)

            def fill(i, carry):
                ldst[pl.ds(i * 16, 16)] = tvec
                return carry

            lax.fori_loop(0, et // 16 + 1, fill, 0)

            def scan(g, off):
                d = dst_raw[pl.ds(g * 16, 16)]
                m = (d >= lo) & (d < lo + chunk)
                plsc.store_compressed(ldst.at[pl.ds(off, 16)], d - lo, mask=m)
                return off + jnp.max(plsc.all_reduce_population_count(m))

            m_cnt = lax.fori_loop(0, et // 16, scan, jnp.int32(0))
            nb = (m_cnt + CB - 1) // CB

            for src_r, dst_r in z_descs():
                pltpu.make_async_copy(src_r, dst_r, zsem).wait()
            plsc.subcore_barrier()

            def c_issue(j, b):
                pltpu.async_copy(ones, cnt.at[ldst.at[pl.ds(j * CB, CB)]],
                                 csem[b], add=True)

            def c_wait(b):
                pltpu.make_async_copy(
                    ones, cnt.at[ldst.at[pl.ds(0, CB)]], csem[b]).wait()

            def quad(i4, carry):
                for b in range(NBUF):
                    j = i4 * NBUF + b

                    @pl.when(j < nb)
                    def _():
                        @pl.when(j >= NBUF)
                        def _():
                            c_wait(b)
                        c_issue(j, b)
                return carry

            lax.fori_loop(0, (nb + NBUF - 1) // NBUF, quad, 0)
            for b in range(NBUF):
                @pl.when(b < nb)
                def _():
                    c_wait(b)
            plsc.subcore_barrier()

            ob = lo + s * wr
            pltpu.sync_copy(cnt.at[pl.ds(s * wr, wr)],
                            outs[t].at[pl.ds(ob, wr)])
            if t < n_types - 1:
                plsc.subcore_barrier()

    f = pl.kernel(
        body, out_type=out_type, mesh=mesh, scratch_types=scratch,
        compiler_params=pltpu.CompilerParams(**_CPARAMS))
    _CNT_CACHE[key] = f
    return f


def _pad_edges(ei, n_edges_pad):
    e = ei.shape[1]
    pad = n_edges_pad - e
    src = jnp.concatenate([ei[0].astype(I32), jnp.zeros((pad,), I32)])
    dst = jnp.concatenate([ei[1].astype(I32), jnp.full((pad,), -1, I32)])
    return src, dst


def _mm_bias(x, w, b):
    n, kd = x.shape
    m = w.shape[1]
    grid = _cdiv(n, BR)

    def body(x_ref, w_ref, b_ref, o_ref):
        o_ref[...] = (
            jnp.dot(x_ref[...], w_ref[...], preferred_element_type=F32)
            + b_ref[...])

    return pl.pallas_call(
        body,
        grid=(grid,),
        in_specs=[
            pl.BlockSpec((BR, kd), lambda i: (i, 0)),
            pl.BlockSpec((kd, m), lambda i: (0, 0)),
            pl.BlockSpec((1, m), lambda i: (0, 0)),
        ],
        out_specs=pl.BlockSpec((BR, m), lambda i: (i, 0)),
        out_shape=jax.ShapeDtypeStruct((n, m), F32),
    )(x, w, b)


def _combine(h, sums, cnts, wl_stack, wr_sum, blm, g, b):
    n = h.shape[0]
    k = len(sums)
    grid = _cdiv(n, BR)

    def body(*refs):
        h_ref = refs[0]
        s_refs = refs[1:1 + k]
        c_refs = refs[1 + k:1 + 2 * k]
        wl_ref, wr_ref, blm_ref, g_ref, b_ref, o_ref = refs[1 + 2 * k:]
        hv = h_ref[...]
        acc = jnp.dot(hv, wr_ref[...], preferred_element_type=F32)
        for i in range(k):
            cntv = c_refs[i][...][:, 0:1]
            recip = 1.0 / jnp.maximum(cntv, 1.0)
            acc = acc + jnp.dot(s_refs[i][...] * recip, wl_ref[i],
                                preferred_element_type=F32)
        x = acc * (1.0 / k) + blm_ref[...]
        mu = jnp.mean(x, axis=-1, keepdims=True)
        var = jnp.mean((x - mu) ** 2, axis=-1, keepdims=True)
        xn = (x - mu) * lax.rsqrt(var + 1e-5) * g_ref[...] + b_ref[...]
        o_ref[...] = xn + hv

    in_specs = [pl.BlockSpec((BR, HID), lambda i: (i, 0))]
    in_specs += [pl.BlockSpec((BR, HID), lambda i: (i, 0))] * k
    in_specs += [pl.BlockSpec((BR, 16), lambda i: (i, 0))] * k
    in_specs += [
        pl.BlockSpec((k, HID, HID), lambda i: (0, 0, 0)),
        pl.BlockSpec((HID, HID), lambda i: (0, 0)),
        pl.BlockSpec((1, HID), lambda i: (0, 0)),
        pl.BlockSpec((1, HID), lambda i: (0, 0)),
        pl.BlockSpec((1, HID), lambda i: (0, 0)),
    ]
    return pl.pallas_call(
        body,
        grid=(grid,),
        in_specs=in_specs,
        out_specs=pl.BlockSpec((BR, HID), lambda i: (i, 0)),
        out_shape=jax.ShapeDtypeStruct((n, HID), F32),
    )(h, *sums, *cnts, wl_stack, wr_sum, blm, g, b)


def kernel(x_occ, x_chord, x_sec, ei_next, ei_prev, ei_inst, ei_inst_rev,
           ei_in_sec, ei_sec_rev, ei_next_sec, Wp_occ, bp_occ, Wp_chord,
           bp_chord, Wp_sec, bp_sec, Wl, bl, Wr, ln_g, ln_b, Wc, bc):
    n = {'occ': x_occ.shape[0], 'chord': x_chord.shape[0],
         'sec': x_sec.shape[0]}
    meta = [('occ', 'occ'), ('occ', 'occ'), ('occ', 'chord'),
            ('chord', 'occ'), ('occ', 'sec'), ('sec', 'occ'), ('sec', 'sec')]
    eis = [ei_next, ei_prev, ei_inst, ei_inst_rev, ei_in_sec, ei_sec_rev,
           ei_next_sec]
    incoming = {'occ': [0, 1, 3, 5], 'chord': [2], 'sec': [4, 6]}
    num_layers = Wl.shape[0]

    h = {'occ': _mm_bias(x_occ, Wp_occ, bp_occ[None]),
         'chord': _mm_bias(x_chord, Wp_chord, bp_chord[None]),
         'sec': _mm_bias(x_sec, Wp_sec, bp_sec[None])}

    seg = []
    eip = []
    for i, (st, dt) in enumerate(meta):
        f, et, n_out = _make_seg_sum(n[st], n[dt], eis[i].shape[1])
        seg.append(f)
        eip.append(_pad_edges(eis[i], NS * et))

    cfg = tuple((n[dt], eis[i].shape[1]) for i, (st, dt) in enumerate(meta))
    fcnt = _make_counts(cfg)
    cnts = fcnt(*[eip[i][1] for i in range(len(meta))])
    cnts = list(cnts) if isinstance(cnts, (tuple, list)) else [cnts]

    for l in range(num_layers):
        sums = {}
        for i, (st, dt) in enumerate(meta):
            out = seg[i](h[st], eip[i][0], eip[i][1])
            sums[i] = out[0] if isinstance(out, (tuple, list)) else out
        h_new = {}
        for nt in ('occ', 'chord', 'sec'):
            idxs = incoming[nt]
            k = len(idxs)
            wl_stack = jnp.stack([Wl[l, i] for i in idxs])
            wr_sum = sum(Wr[l, i] for i in idxs)
            blm = (sum(bl[l, i] for i in idxs) / k)[None]
            h_new[nt] = _combine(
                h[nt], [sums[i] for i in idxs], [cnts[i] for i in idxs],
                wl_stack, wr_sum, blm, ln_g[l][None], ln_b[l][None])
        h = h_new

    return _mm_bias(h['occ'], Wc, bc[None])


# speedup vs baseline: 2.5294x; 1.1447x over previous
import jax
import jax.numpy as jnp
from jax import lax
from jax.experimental import pallas as pl
from jax.experimental.pallas import tpu as pltpu
from jax.experimental.pallas import tpu_sc as plsc

F32 = jnp.float32
I32 = jnp.int32
NC = 2
NS = 16
HID = 128
BR = 512
NBUF = 8
BLK = 32

_MESH = dict(core_axis_name="c", subcore_axis_name="s",
             num_cores=NC, num_subcores=NS)
_CPARAMS = dict(needs_layout_passes=False, use_tc_tiling_on_sc=False)


def _cdiv(a, b):
    return -(-a // b)


def _et_of(n_edges):
    return max(2, _cdiv(n_edges, NS * 128)) * 128


_SEG_CACHE = {}
_SPMEM_BUDGET = 4_700_000


def _seg_geometry(n_dst):
    p = 1
    while True:
        chunk = _cdiv(n_dst, NC * p * 128) * 128
        if (chunk + 128) * 512 <= _SPMEM_BUDGET:
            return p, chunk
        p += 1


_A_MAX = 8576


def _make_seg_sum(n_src, n_dst, n_edges):
    key = (n_src, n_dst, n_edges)
    if key in _SEG_CACHE:
        return _SEG_CACHE[key]

    et = _et_of(n_edges)
    P, chunk = _seg_geometry(n_dst)
    A = chunk + 128
    assert A <= _A_MAX
    n_out = NC * P * chunk
    zr = A // 16
    wr = chunk // 16

    scratch = [
        pltpu.VMEM((et,), I32),
        pltpu.VMEM((et,), I32),
        pltpu.VMEM((et + BLK,), I32),
        pltpu.VMEM((et + BLK,), I32),
        pltpu.VMEM((NBUF, BLK, HID), F32),
        pltpu.VMEM((64, HID), F32),
        pltpu.VMEM_SHARED((A, HID), F32),
    ]
    scratch += [pltpu.SemaphoreType.DMA] * (2 * NBUF + 1)

    mesh = plsc.VectorSubcoreMesh(**_MESH)

    def body(hsrc, src_hbm, dst_hbm, sums_o, src_raw, dst_raw,
             lsrc, ldst, rows, zrow, acc, *sems):
        gsem = sems[:NBUF]
        ssem = sems[NBUF:2 * NBUF]
        zsem = sems[2 * NBUF]

        c = lax.axis_index("c")
        s = lax.axis_index("s")

        zvec = jnp.zeros((16,), F32)

        def init(r, carry):
            for v in range(HID // 16):
                zrow[r, pl.ds(v * 16, 16)] = zvec
            return carry

        lax.fori_loop(0, 64, init, 0)

        base = s * et
        pltpu.sync_copy(src_hbm.at[pl.ds(base, et)], src_raw)
        pltpu.sync_copy(dst_hbm.at[pl.ds(base, et)], dst_raw)

        zb = s * zr
        znf, zrem = divmod(zr, 64)

        def z_descs():
            ds_ = []
            for q in range(znf):
                ds_.append((zrow, acc.at[pl.ds(zb + q * 64, 64)]))
            if zrem:
                ds_.append((zrow.at[pl.ds(0, zrem)],
                            acc.at[pl.ds(zb + znf * 64, zrem)]))
            return ds_

        def g_issue(j, b):
            pltpu.async_copy(
                hsrc.at[lsrc.at[pl.ds(j * BLK, BLK)]], rows.at[b], gsem[b])

        def g_wait(b):
            pltpu.make_async_copy(
                hsrc.at[lsrc.at[pl.ds(0, BLK)]], rows.at[b], gsem[b]).wait()

        def s_issue(j, b):
            pltpu.async_copy(rows.at[b],
                             acc.at[ldst.at[pl.ds(j * BLK, BLK)]],
                             ssem[b], add=True)

        def s_wait(b):
            pltpu.make_async_copy(
                rows.at[b], acc.at[ldst.at[pl.ds(0, BLK)]], ssem[b]).wait()

        for p in range(P):
            ri = c * P + p
            lo = ri * chunk

            for src_r, dst_r in z_descs():
                pltpu.async_copy(src_r, dst_r, zsem)

            zivec = jnp.zeros((16,), I32)
            tvec = jnp.full((16,), chunk, I32)

            def scan(g, off):
                d = dst_raw[pl.ds(g * 16, 16)]
                sv = src_raw[pl.ds(g * 16, 16)]
                m = (d >= lo) & (d < lo + chunk)
                plsc.store_compressed(lsrc.at[pl.ds(off, 16)], sv, mask=m)
                plsc.store_compressed(ldst.at[pl.ds(off, 16)], d - lo, mask=m)
                return off + jnp.max(plsc.all_reduce_population_count(m))

            m_cnt = lax.fori_loop(0, et // 16, scan, jnp.int32(0))
            nb = (m_cnt + BLK - 1) // BLK

            for g in range(BLK // 16):
                lsrc[pl.ds(m_cnt + g * 16, 16)] = zivec
                ldst[pl.ds(m_cnt + g * 16, 16)] = tvec

            for src_r, dst_r in z_descs():
                pltpu.make_async_copy(src_r, dst_r, zsem).wait()
            plsc.subcore_barrier()

            @pl.when(nb > 0)
            def _():
                g_issue(0, 0)

            def quad(i4, carry):
                for b in range(NBUF):
                    j = i4 * NBUF + b

                    @pl.when(j < nb)
                    def _():
                        jn = j + 1
                        bn = (b + 1) % NBUF

                        @pl.when(jn < nb)
                        def _():
                            @pl.when(jn >= NBUF)
                            def _():
                                s_wait(bn)
                            g_issue(jn, bn)

                        g_wait(b)
                        s_issue(j, b)
                return carry

            lax.fori_loop(0, (nb + NBUF - 1) // NBUF, quad, 0)
            for b in range(NBUF):
                @pl.when(b < nb)
                def _():
                    s_wait(b)
            plsc.subcore_barrier()

            ob = lo + s * wr
            pltpu.sync_copy(acc.at[pl.ds(s * wr, wr)],
                            sums_o.at[pl.ds(ob, wr)])
            if p < P - 1:
                plsc.subcore_barrier()

    f = pl.kernel(
        body,
        out_type=[jax.ShapeDtypeStruct((n_out, HID), F32)],
        mesh=mesh, scratch_types=scratch,
        compiler_params=pltpu.CompilerParams(**_CPARAMS))
    _SEG_CACHE[key] = (f, et, n_out)
    return _SEG_CACHE[key]


_CNT_CACHE = {}


def _make_counts(configs):
    key = tuple(configs)
    if key in _CNT_CACHE:
        return _CNT_CACHE[key]

    geo = []
    for n_dst, n_edges in configs:
        et = _et_of(n_edges)
        chunk = _cdiv(n_dst, NC * 128) * 128
        geo.append((et, chunk))
    et_max = max(g[0] for g in geo)
    a_max = max(g[1] for g in geo) + 128
    CB = 128

    out_type = [jax.ShapeDtypeStruct((NC * g[1], 16), F32) for g in geo]
    scratch = [
        pltpu.VMEM((et_max,), I32),
        pltpu.VMEM((et_max + 16,), I32),
        pltpu.VMEM((CB, 16), F32),
        pltpu.VMEM((CB, 16), F32),
        pltpu.VMEM_SHARED((a_max, 16), F32),
    ]
    scratch += [pltpu.SemaphoreType.DMA] * (NBUF + 1)

    mesh = plsc.VectorSubcoreMesh(**_MESH)
    n_types = len(configs)

    def body(*refs):
        dst_hbms = refs[:n_types]
        outs = refs[n_types:2 * n_types]
        dst_raw, ldst, ones, zcnt, cnt = refs[2 * n_types:2 * n_types + 5]
        sems = refs[2 * n_types + 5:]
        csem = sems[:NBUF]
        zsem = sems[NBUF]

        c = lax.axis_index("c")
        s = lax.axis_index("s")

        zvec = jnp.zeros((16,), F32)
        ovec = jnp.ones((16,), F32)

        def init(r, carry):
            ones[r, :] = ovec
            zcnt[r, :] = zvec
            return carry

        lax.fori_loop(0, CB, init, 0)

        for t in range(n_types):
            et, chunk = geo[t]
            A = chunk + 128
            zr = A // 16
            wr = chunk // 16
            lo = c * chunk

            znf, zrem = divmod(zr, CB)
            zb = s * zr

            def z_descs():
                ds_ = []
                for q in range(znf):
                    ds_.append((zcnt, cnt.at[pl.ds(zb + q * CB, CB)]))
                if zrem:
                    ds_.append((zcnt.at[pl.ds(0, zrem)],
                                cnt.at[pl.ds(zb + znf * CB, zrem)]))
                return ds_

            for src_r, dst_r in z_descs():
                pltpu.async_copy(src_r, dst_r, zsem)

            base = s * et
            pltpu.sync_copy(dst_hbms[t].at[pl.ds(base, et)],
                            dst_raw.at[pl.ds(0, et)])

            tvec = jnp.full((16,), chunk, I32)

            def fill(i, carry):
                ldst[pl.ds(i * 16, 16)] = tvec
                return carry

            lax.fori_loop(0, et // 16 + 1, fill, 0)

            def scan(g, off):
                d = dst_raw[pl.ds(g * 16, 16)]
                m = (d >= lo) & (d < lo + chunk)
                plsc.store_compressed(ldst.at[pl.ds(off, 16)], d - lo, mask=m)
                return off + jnp.max(plsc.all_reduce_population_count(m))

            m_cnt = lax.fori_loop(0, et // 16, scan, jnp.int32(0))
            nb = (m_cnt + CB - 1) // CB

            for src_r, dst_r in z_descs():
                pltpu.make_async_copy(src_r, dst_r, zsem).wait()
            plsc.subcore_barrier()

            def c_issue(j, b):
                pltpu.async_copy(ones, cnt.at[ldst.at[pl.ds(j * CB, CB)]],
                                 csem[b], add=True)

            def c_wait(b):
                pltpu.make_async_copy(
                    ones, cnt.at[ldst.at[pl.ds(0, CB)]], csem[b]).wait()

            def quad(i4, carry):
                for b in range(NBUF):
                    j = i4 * NBUF + b

                    @pl.when(j < nb)
                    def _():
                        @pl.when(j >= NBUF)
                        def _():
                            c_wait(b)
                        c_issue(j, b)
                return carry

            lax.fori_loop(0, (nb + NBUF - 1) // NBUF, quad, 0)
            for b in range(NBUF):
                @pl.when(b < nb)
                def _():
                    c_wait(b)
            plsc.subcore_barrier()

            ob = lo + s * wr
            pltpu.sync_copy(cnt.at[pl.ds(s * wr, wr)],
                            outs[t].at[pl.ds(ob, wr)])
            if t < n_types - 1:
                plsc.subcore_barrier()

    f = pl.kernel(
        body, out_type=out_type, mesh=mesh, scratch_types=scratch,
        compiler_params=pltpu.CompilerParams(**_CPARAMS))
    _CNT_CACHE[key] = f
    return f


def _pad_edges(ei, n_edges_pad):
    e = ei.shape[1]
    pad = n_edges_pad - e
    src = jnp.concatenate([ei[0].astype(I32), jnp.zeros((pad,), I32)])
    dst = jnp.concatenate([ei[1].astype(I32), jnp.full((pad,), -1, I32)])
    return src, dst


def _mm_bias(x, w, b):
    n, kd = x.shape
    m = w.shape[1]
    grid = _cdiv(n, BR)

    def body(x_ref, w_ref, b_ref, o_ref):
        o_ref[...] = (
            jnp.dot(x_ref[...], w_ref[...], preferred_element_type=F32)
            + b_ref[...])

    return pl.pallas_call(
        body,
        grid=(grid,),
        in_specs=[
            pl.BlockSpec((BR, kd), lambda i: (i, 0)),
            pl.BlockSpec((kd, m), lambda i: (0, 0)),
            pl.BlockSpec((1, m), lambda i: (0, 0)),
        ],
        out_specs=pl.BlockSpec((BR, m), lambda i: (i, 0)),
        out_shape=jax.ShapeDtypeStruct((n, m), F32),
    )(x, w, b)


def _combine(h, sums, cnts, wl_stack, wr_sum, blm, g, b):
    n = h.shape[0]
    k = len(sums)
    grid = _cdiv(n, BR)

    def body(*refs):
        h_ref = refs[0]
        s_refs = refs[1:1 + k]
        c_refs = refs[1 + k:1 + 2 * k]
        wl_ref, wr_ref, blm_ref, g_ref, b_ref, o_ref = refs[1 + 2 * k:]
        hv = h_ref[...]
        acc = jnp.dot(hv, wr_ref[...], preferred_element_type=F32)
        for i in range(k):
            cntv = c_refs[i][...][:, 0:1]
            recip = 1.0 / jnp.maximum(cntv, 1.0)
            acc = acc + jnp.dot(s_refs[i][...] * recip, wl_ref[i],
                                preferred_element_type=F32)
        x = acc * (1.0 / k) + blm_ref[...]
        mu = jnp.mean(x, axis=-1, keepdims=True)
        var = jnp.mean((x - mu) ** 2, axis=-1, keepdims=True)
        xn = (x - mu) * lax.rsqrt(var + 1e-5) * g_ref[...] + b_ref[...]
        o_ref[...] = xn + hv

    in_specs = [pl.BlockSpec((BR, HID), lambda i: (i, 0))]
    in_specs += [pl.BlockSpec((BR, HID), lambda i: (i, 0))] * k
    in_specs += [pl.BlockSpec((BR, 16), lambda i: (i, 0))] * k
    in_specs += [
        pl.BlockSpec((k, HID, HID), lambda i: (0, 0, 0)),
        pl.BlockSpec((HID, HID), lambda i: (0, 0)),
        pl.BlockSpec((1, HID), lambda i: (0, 0)),
        pl.BlockSpec((1, HID), lambda i: (0, 0)),
        pl.BlockSpec((1, HID), lambda i: (0, 0)),
    ]
    return pl.pallas_call(
        body,
        grid=(grid,),
        in_specs=in_specs,
        out_specs=pl.BlockSpec((BR, HID), lambda i: (i, 0)),
        out_shape=jax.ShapeDtypeStruct((n, HID), F32),
    )(h, *sums, *cnts, wl_stack, wr_sum, blm, g, b)


def kernel(x_occ, x_chord, x_sec, ei_next, ei_prev, ei_inst, ei_inst_rev,
           ei_in_sec, ei_sec_rev, ei_next_sec, Wp_occ, bp_occ, Wp_chord,
           bp_chord, Wp_sec, bp_sec, Wl, bl, Wr, ln_g, ln_b, Wc, bc):
    n = {'occ': x_occ.shape[0], 'chord': x_chord.shape[0],
         'sec': x_sec.shape[0]}
    meta = [('occ', 'occ'), ('occ', 'occ'), ('occ', 'chord'),
            ('chord', 'occ'), ('occ', 'sec'), ('sec', 'occ'), ('sec', 'sec')]
    eis = [ei_next, ei_prev, ei_inst, ei_inst_rev, ei_in_sec, ei_sec_rev,
           ei_next_sec]
    incoming = {'occ': [0, 1, 3, 5], 'chord': [2], 'sec': [4, 6]}
    num_layers = Wl.shape[0]

    h = {'occ': _mm_bias(x_occ, Wp_occ, bp_occ[None]),
         'chord': _mm_bias(x_chord, Wp_chord, bp_chord[None]),
         'sec': _mm_bias(x_sec, Wp_sec, bp_sec[None])}

    seg = []
    eip = []
    for i, (st, dt) in enumerate(meta):
        f, et, n_out = _make_seg_sum(n[st], n[dt], eis[i].shape[1])
        seg.append(f)
        eip.append(_pad_edges(eis[i], NS * et))

    cfg = tuple((n[dt], eis[i].shape[1]) for i, (st, dt) in enumerate(meta))
    fcnt = _make_counts(cfg)
    cnts = fcnt(*[eip[i][1] for i in range(len(meta))])
    cnts = list(cnts) if isinstance(cnts, (tuple, list)) else [cnts]

    for l in range(num_layers):
        sums = {}
        for i, (st, dt) in enumerate(meta):
            out = seg[i](h[st], eip[i][0], eip[i][1])
            sums[i] = out[0] if isinstance(out, (tuple, list)) else out
        h_new = {}
        for nt in ('occ', 'chord', 'sec'):
            idxs = incoming[nt]
            k = len(idxs)
            wl_stack = jnp.stack([Wl[l, i] for i in idxs])
            wr_sum = sum(Wr[l, i] for i in idxs)
            blm = (sum(bl[l, i] for i in idxs) / k)[None]
            h_new[nt] = _combine(
                h[nt], [sums[i] for i in idxs], [cnts[i] for i in idxs],
                wl_stack, wr_sum, blm, ln_g[l][None], ln_b[l][None])
        h = h_new

    return _mm_bias(h['occ'], Wc, bc[None])


# speedup vs baseline: 2.6986x; 1.0669x over previous
import jax
import jax.numpy as jnp
from jax import lax
from jax.experimental import pallas as pl
from jax.experimental.pallas import tpu as pltpu
from jax.experimental.pallas import tpu_sc as plsc

F32 = jnp.float32
I32 = jnp.int32
NC = 2
NS = 16
HID = 128
BR = 1024
NBUF = 8
BLK = 32

_MESH = dict(core_axis_name="c", subcore_axis_name="s",
             num_cores=NC, num_subcores=NS)
_CPARAMS = dict(needs_layout_passes=False, use_tc_tiling_on_sc=False)


def _cdiv(a, b):
    return -(-a // b)


def _et_of(n_edges):
    return max(2, _cdiv(n_edges, NS * 128)) * 128


_SEG_CACHE = {}
_SPMEM_BUDGET = 4_700_000


def _seg_geometry(n_dst):
    p = 1
    while True:
        chunk = _cdiv(n_dst, NC * p * 128) * 128
        if (chunk + 128) * 512 <= _SPMEM_BUDGET:
            return p, chunk
        p += 1


_A_MAX = 8576


def _make_seg_sum(n_src, n_dst, n_edges):
    key = (n_src, n_dst, n_edges)
    if key in _SEG_CACHE:
        return _SEG_CACHE[key]

    et = _et_of(n_edges)
    P, chunk = _seg_geometry(n_dst)
    A = chunk + 128
    assert A <= _A_MAX
    n_out = NC * P * chunk
    zr = A // 16
    wr = chunk // 16

    scratch = [
        pltpu.VMEM((et,), I32),
        pltpu.VMEM((et,), I32),
        pltpu.VMEM((et + BLK,), I32),
        pltpu.VMEM((et + BLK,), I32),
        pltpu.VMEM((NBUF, BLK, HID), F32),
        pltpu.VMEM((64, HID), F32),
        pltpu.VMEM_SHARED((A, HID), F32),
    ]
    scratch += [pltpu.SemaphoreType.DMA] * (2 * NBUF + 1)

    mesh = plsc.VectorSubcoreMesh(**_MESH)

    def body(hsrc, src_hbm, dst_hbm, sums_o, src_raw, dst_raw,
             lsrc, ldst, rows, zrow, acc, *sems):
        gsem = sems[:NBUF]
        ssem = sems[NBUF:2 * NBUF]
        zsem = sems[2 * NBUF]

        c = lax.axis_index("c")
        s = lax.axis_index("s")

        zvec = jnp.zeros((16,), F32)

        def init(r, carry):
            for v in range(HID // 16):
                zrow[r, pl.ds(v * 16, 16)] = zvec
            return carry

        lax.fori_loop(0, 64, init, 0)

        base = s * et
        pltpu.sync_copy(src_hbm.at[pl.ds(base, et)], src_raw)
        pltpu.sync_copy(dst_hbm.at[pl.ds(base, et)], dst_raw)

        zb = s * zr
        znf, zrem = divmod(zr, 64)

        def z_descs():
            ds_ = []
            for q in range(znf):
                ds_.append((zrow, acc.at[pl.ds(zb + q * 64, 64)]))
            if zrem:
                ds_.append((zrow.at[pl.ds(0, zrem)],
                            acc.at[pl.ds(zb + znf * 64, zrem)]))
            return ds_

        def g_issue(j, b):
            pltpu.async_copy(
                hsrc.at[lsrc.at[pl.ds(j * BLK, BLK)]], rows.at[b], gsem[b])

        def g_wait(b):
            pltpu.make_async_copy(
                hsrc.at[lsrc.at[pl.ds(0, BLK)]], rows.at[b], gsem[b]).wait()

        def s_issue(j, b):
            pltpu.async_copy(rows.at[b],
                             acc.at[ldst.at[pl.ds(j * BLK, BLK)]],
                             ssem[b], add=True)

        def s_wait(b):
            pltpu.make_async_copy(
                rows.at[b], acc.at[ldst.at[pl.ds(0, BLK)]], ssem[b]).wait()

        for p in range(P):
            ri = c * P + p
            lo = ri * chunk

            for src_r, dst_r in z_descs():
                pltpu.async_copy(src_r, dst_r, zsem)

            zivec = jnp.zeros((16,), I32)
            tvec = jnp.full((16,), chunk, I32)

            def scan(g, off):
                d = dst_raw[pl.ds(g * 16, 16)]
                sv = src_raw[pl.ds(g * 16, 16)]
                m = (d >= lo) & (d < lo + chunk)
                plsc.store_compressed(lsrc.at[pl.ds(off, 16)], sv, mask=m)
                plsc.store_compressed(ldst.at[pl.ds(off, 16)], d - lo, mask=m)
                return off + jnp.max(plsc.all_reduce_population_count(m))

            m_cnt = lax.fori_loop(0, et // 16, scan, jnp.int32(0))
            nb = (m_cnt + BLK - 1) // BLK

            for g in range(BLK // 16):
                lsrc[pl.ds(m_cnt + g * 16, 16)] = zivec
                ldst[pl.ds(m_cnt + g * 16, 16)] = tvec

            for src_r, dst_r in z_descs():
                pltpu.make_async_copy(src_r, dst_r, zsem).wait()
            plsc.subcore_barrier()

            @pl.when(nb > 0)
            def _():
                g_issue(0, 0)

            def quad(i4, carry):
                for b in range(NBUF):
                    j = i4 * NBUF + b

                    @pl.when(j < nb)
                    def _():
                        jn = j + 1
                        bn = (b + 1) % NBUF

                        @pl.when(jn < nb)
                        def _():
                            @pl.when(jn >= NBUF)
                            def _():
                                s_wait(bn)
                            g_issue(jn, bn)

                        g_wait(b)
                        s_issue(j, b)
                return carry

            lax.fori_loop(0, (nb + NBUF - 1) // NBUF, quad, 0)
            for b in range(NBUF):
                @pl.when(b < nb)
                def _():
                    s_wait(b)
            plsc.subcore_barrier()

            ob = lo + s * wr
            pltpu.sync_copy(acc.at[pl.ds(s * wr, wr)],
                            sums_o.at[pl.ds(ob, wr)])
            if p < P - 1:
                plsc.subcore_barrier()

    f = pl.kernel(
        body,
        out_type=[jax.ShapeDtypeStruct((n_out, HID), F32)],
        mesh=mesh, scratch_types=scratch,
        compiler_params=pltpu.CompilerParams(**_CPARAMS))
    _SEG_CACHE[key] = (f, et, n_out)
    return _SEG_CACHE[key]


_CNT_CACHE = {}


def _make_counts(configs):
    key = tuple(configs)
    if key in _CNT_CACHE:
        return _CNT_CACHE[key]

    geo = []
    for n_dst, n_edges in configs:
        et = _et_of(n_edges)
        chunk = _cdiv(n_dst, NC * 128) * 128
        geo.append((et, chunk))
    et_max = max(g[0] for g in geo)
    a_max = max(g[1] for g in geo) + 128
    CB = 128

    out_type = [jax.ShapeDtypeStruct((NC * g[1], 16), F32) for g in geo]
    scratch = [
        pltpu.VMEM((et_max,), I32),
        pltpu.VMEM((et_max + 16,), I32),
        pltpu.VMEM((CB, 16), F32),
        pltpu.VMEM((CB, 16), F32),
        pltpu.VMEM_SHARED((a_max, 16), F32),
    ]
    scratch += [pltpu.SemaphoreType.DMA] * (NBUF + 1)

    mesh = plsc.VectorSubcoreMesh(**_MESH)
    n_types = len(configs)

    def body(*refs):
        dst_hbms = refs[:n_types]
        outs = refs[n_types:2 * n_types]
        dst_raw, ldst, ones, zcnt, cnt = refs[2 * n_types:2 * n_types + 5]
        sems = refs[2 * n_types + 5:]
        csem = sems[:NBUF]
        zsem = sems[NBUF]

        c = lax.axis_index("c")
        s = lax.axis_index("s")

        zvec = jnp.zeros((16,), F32)
        ovec = jnp.ones((16,), F32)

        def init(r, carry):
            ones[r, :] = ovec
            zcnt[r, :] = zvec
            return carry

        lax.fori_loop(0, CB, init, 0)

        for t in range(n_types):
            et, chunk = geo[t]
            A = chunk + 128
            zr = A // 16
            wr = chunk // 16
            lo = c * chunk

            znf, zrem = divmod(zr, CB)
            zb = s * zr

            def z_descs():
                ds_ = []
                for q in range(znf):
                    ds_.append((zcnt, cnt.at[pl.ds(zb + q * CB, CB)]))
                if zrem:
                    ds_.append((zcnt.at[pl.ds(0, zrem)],
                                cnt.at[pl.ds(zb + znf * CB, zrem)]))
                return ds_

            for src_r, dst_r in z_descs():
                pltpu.async_copy(src_r, dst_r, zsem)

            base = s * et
            pltpu.sync_copy(dst_hbms[t].at[pl.ds(base, et)],
                            dst_raw.at[pl.ds(0, et)])

            tvec = jnp.full((16,), chunk, I32)

            def fill(i, carry):
                ldst[pl.ds(i * 16, 16)] = tvec
                return carry

            lax.fori_loop(0, et // 16 + 1, fill, 0)

            def scan(g, off):
                d = dst_raw[pl.ds(g * 16, 16)]
                m = (d >= lo) & (d < lo + chunk)
                plsc.store_compressed(ldst.at[pl.ds(off, 16)], d - lo, mask=m)
                return off + jnp.max(plsc.all_reduce_population_count(m))

            m_cnt = lax.fori_loop(0, et // 16, scan, jnp.int32(0))
            nb = (m_cnt + CB - 1) // CB

            for src_r, dst_r in z_descs():
                pltpu.make_async_copy(src_r, dst_r, zsem).wait()
            plsc.subcore_barrier()

            def c_issue(j, b):
                pltpu.async_copy(ones, cnt.at[ldst.at[pl.ds(j * CB, CB)]],
                                 csem[b], add=True)

            def c_wait(b):
                pltpu.make_async_copy(
                    ones, cnt.at[ldst.at[pl.ds(0, CB)]], csem[b]).wait()

            def quad(i4, carry):
                for b in range(NBUF):
                    j = i4 * NBUF + b

                    @pl.when(j < nb)
                    def _():
                        @pl.when(j >= NBUF)
                        def _():
                            c_wait(b)
                        c_issue(j, b)
                return carry

            lax.fori_loop(0, (nb + NBUF - 1) // NBUF, quad, 0)
            for b in range(NBUF):
                @pl.when(b < nb)
                def _():
                    c_wait(b)
            plsc.subcore_barrier()

            ob = lo + s * wr
            pltpu.sync_copy(cnt.at[pl.ds(s * wr, wr)],
                            outs[t].at[pl.ds(ob, wr)])
            if t < n_types - 1:
                plsc.subcore_barrier()

    f = pl.kernel(
        body, out_type=out_type, mesh=mesh, scratch_types=scratch,
        compiler_params=pltpu.CompilerParams(**_CPARAMS))
    _CNT_CACHE[key] = f
    return f


def _pad_edges(ei, n_edges_pad):
    e = ei.shape[1]
    pad = n_edges_pad - e
    src = jnp.concatenate([ei[0].astype(I32), jnp.zeros((pad,), I32)])
    dst = jnp.concatenate([ei[1].astype(I32), jnp.full((pad,), -1, I32)])
    return src, dst


def _mm_bias(x, w, b):
    n, kd = x.shape
    m = w.shape[1]
    grid = _cdiv(n, BR)

    def body(x_ref, w_ref, b_ref, o_ref):
        o_ref[...] = (
            jnp.dot(x_ref[...], w_ref[...], preferred_element_type=F32)
            + b_ref[...])

    return pl.pallas_call(
        body,
        grid=(grid,),
        in_specs=[
            pl.BlockSpec((BR, kd), lambda i: (i, 0)),
            pl.BlockSpec((kd, m), lambda i: (0, 0)),
            pl.BlockSpec((1, m), lambda i: (0, 0)),
        ],
        out_specs=pl.BlockSpec((BR, m), lambda i: (i, 0)),
        out_shape=jax.ShapeDtypeStruct((n, m), F32),
    )(x, w, b)


def _combine(h, sums, cnts, wl_stack, wr_sum, blm, g, b):
    n = h.shape[0]
    k = len(sums)
    grid = _cdiv(n, BR)

    def body(*refs):
        h_ref = refs[0]
        s_refs = refs[1:1 + k]
        c_refs = refs[1 + k:1 + 2 * k]
        wl_ref, wr_ref, blm_ref, g_ref, b_ref, o_ref = refs[1 + 2 * k:]
        hv = h_ref[...]
        acc = jnp.dot(hv, wr_ref[...], preferred_element_type=F32)
        for i in range(k):
            cntv = c_refs[i][...][:, 0:1]
            recip = 1.0 / jnp.maximum(cntv, 1.0)
            acc = acc + jnp.dot(s_refs[i][...] * recip, wl_ref[i],
                                preferred_element_type=F32)
        x = acc * (1.0 / k) + blm_ref[...]
        mu = jnp.mean(x, axis=-1, keepdims=True)
        var = jnp.mean((x - mu) ** 2, axis=-1, keepdims=True)
        xn = (x - mu) * lax.rsqrt(var + 1e-5) * g_ref[...] + b_ref[...]
        o_ref[...] = xn + hv

    in_specs = [pl.BlockSpec((BR, HID), lambda i: (i, 0))]
    in_specs += [pl.BlockSpec((BR, HID), lambda i: (i, 0))] * k
    in_specs += [pl.BlockSpec((BR, 16), lambda i: (i, 0))] * k
    in_specs += [
        pl.BlockSpec((k, HID, HID), lambda i: (0, 0, 0)),
        pl.BlockSpec((HID, HID), lambda i: (0, 0)),
        pl.BlockSpec((1, HID), lambda i: (0, 0)),
        pl.BlockSpec((1, HID), lambda i: (0, 0)),
        pl.BlockSpec((1, HID), lambda i: (0, 0)),
    ]
    return pl.pallas_call(
        body,
        grid=(grid,),
        in_specs=in_specs,
        out_specs=pl.BlockSpec((BR, HID), lambda i: (i, 0)),
        out_shape=jax.ShapeDtypeStruct((n, HID), F32),
    )(h, *sums, *cnts, wl_stack, wr_sum, blm, g, b)


def kernel(x_occ, x_chord, x_sec, ei_next, ei_prev, ei_inst, ei_inst_rev,
           ei_in_sec, ei_sec_rev, ei_next_sec, Wp_occ, bp_occ, Wp_chord,
           bp_chord, Wp_sec, bp_sec, Wl, bl, Wr, ln_g, ln_b, Wc, bc):
    n = {'occ': x_occ.shape[0], 'chord': x_chord.shape[0],
         'sec': x_sec.shape[0]}
    meta = [('occ', 'occ'), ('occ', 'occ'), ('occ', 'chord'),
            ('chord', 'occ'), ('occ', 'sec'), ('sec', 'occ'), ('sec', 'sec')]
    eis = [ei_next, ei_prev, ei_inst, ei_inst_rev, ei_in_sec, ei_sec_rev,
           ei_next_sec]
    incoming = {'occ': [0, 1, 3, 5], 'chord': [2], 'sec': [4, 6]}
    num_layers = Wl.shape[0]

    h = {'occ': _mm_bias(x_occ, Wp_occ, bp_occ[None]),
         'chord': _mm_bias(x_chord, Wp_chord, bp_chord[None]),
         'sec': _mm_bias(x_sec, Wp_sec, bp_sec[None])}

    seg = []
    eip = []
    for i, (st, dt) in enumerate(meta):
        f, et, n_out = _make_seg_sum(n[st], n[dt], eis[i].shape[1])
        seg.append(f)
        eip.append(_pad_edges(eis[i], NS * et))

    cfg = tuple((n[dt], eis[i].shape[1]) for i, (st, dt) in enumerate(meta))
    fcnt = _make_counts(cfg)
    cnts = fcnt(*[eip[i][1] for i in range(len(meta))])
    cnts = list(cnts) if isinstance(cnts, (tuple, list)) else [cnts]

    for l in range(num_layers):
        sums = {}
        for i, (st, dt) in enumerate(meta):
            out = seg[i](h[st], eip[i][0], eip[i][1])
            sums[i] = out[0] if isinstance(out, (tuple, list)) else out
        h_new = {}
        for nt in ('occ', 'chord', 'sec'):
            idxs = incoming[nt]
            k = len(idxs)
            wl_stack = jnp.stack([Wl[l, i] for i in idxs])
            wr_sum = sum(Wr[l, i] for i in idxs)
            blm = (sum(bl[l, i] for i in idxs) / k)[None]
            h_new[nt] = _combine(
                h[nt], [sums[i] for i in idxs], [cnts[i] for i in idxs],
                wl_stack, wr_sum, blm, ln_g[l][None], ln_b[l][None])
        h = h_new

    return _mm_bias(h['occ'], Wc, bc[None])


# speedup vs baseline: 2.7657x; 1.0249x over previous
import jax
import jax.numpy as jnp
from jax import lax
from jax.experimental import pallas as pl
from jax.experimental.pallas import tpu as pltpu
from jax.experimental.pallas import tpu_sc as plsc

F32 = jnp.float32
I32 = jnp.int32
NC = 2
NS = 16
HID = 128
BR = 2048
NBUF = 8
BLK = 32

_MESH = dict(core_axis_name="c", subcore_axis_name="s",
             num_cores=NC, num_subcores=NS)
_CPARAMS = dict(needs_layout_passes=False, use_tc_tiling_on_sc=False)


def _cdiv(a, b):
    return -(-a // b)


def _et_of(n_edges):
    return max(2, _cdiv(n_edges, NS * 128)) * 128


_SEG_CACHE = {}
_SPMEM_BUDGET = 4_700_000


def _seg_geometry(n_dst):
    p = 1
    while True:
        chunk = _cdiv(n_dst, NC * p * 128) * 128
        if (chunk + 128) * 512 <= _SPMEM_BUDGET:
            return p, chunk
        p += 1


_A_MAX = 8576


def _make_seg_sum(n_src, n_dst, n_edges):
    key = (n_src, n_dst, n_edges)
    if key in _SEG_CACHE:
        return _SEG_CACHE[key]

    et = _et_of(n_edges)
    P, chunk = _seg_geometry(n_dst)
    A = chunk + 128
    assert A <= _A_MAX
    n_out = NC * P * chunk
    zr = A // 16
    wr = chunk // 16

    scratch = [
        pltpu.VMEM((et,), I32),
        pltpu.VMEM((et,), I32),
        pltpu.VMEM((et + BLK,), I32),
        pltpu.VMEM((et + BLK,), I32),
        pltpu.VMEM((NBUF, BLK, HID), F32),
        pltpu.VMEM((64, HID), F32),
        pltpu.VMEM_SHARED((A, HID), F32),
    ]
    scratch += [pltpu.SemaphoreType.DMA] * (2 * NBUF + 1)

    mesh = plsc.VectorSubcoreMesh(**_MESH)

    def body(hsrc, src_hbm, dst_hbm, sums_o, src_raw, dst_raw,
             lsrc, ldst, rows, zrow, acc, *sems):
        gsem = sems[:NBUF]
        ssem = sems[NBUF:2 * NBUF]
        zsem = sems[2 * NBUF]

        c = lax.axis_index("c")
        s = lax.axis_index("s")

        zvec = jnp.zeros((16,), F32)

        def init(r, carry):
            for v in range(HID // 16):
                zrow[r, pl.ds(v * 16, 16)] = zvec
            return carry

        lax.fori_loop(0, 64, init, 0)

        base = s * et
        pltpu.sync_copy(src_hbm.at[pl.ds(base, et)], src_raw)
        pltpu.sync_copy(dst_hbm.at[pl.ds(base, et)], dst_raw)

        zb = s * zr
        znf, zrem = divmod(zr, 64)

        def z_descs():
            ds_ = []
            for q in range(znf):
                ds_.append((zrow, acc.at[pl.ds(zb + q * 64, 64)]))
            if zrem:
                ds_.append((zrow.at[pl.ds(0, zrem)],
                            acc.at[pl.ds(zb + znf * 64, zrem)]))
            return ds_

        def g_issue(j, b):
            pltpu.async_copy(
                hsrc.at[lsrc.at[pl.ds(j * BLK, BLK)]], rows.at[b], gsem[b])

        def g_wait(b):
            pltpu.make_async_copy(
                hsrc.at[lsrc.at[pl.ds(0, BLK)]], rows.at[b], gsem[b]).wait()

        def s_issue(j, b):
            pltpu.async_copy(rows.at[b],
                             acc.at[ldst.at[pl.ds(j * BLK, BLK)]],
                             ssem[b], add=True)

        def s_wait(b):
            pltpu.make_async_copy(
                rows.at[b], acc.at[ldst.at[pl.ds(0, BLK)]], ssem[b]).wait()

        for p in range(P):
            ri = c * P + p
            lo = ri * chunk

            for src_r, dst_r in z_descs():
                pltpu.async_copy(src_r, dst_r, zsem)

            zivec = jnp.zeros((16,), I32)
            tvec = jnp.full((16,), chunk, I32)

            def scan(g, off):
                d = dst_raw[pl.ds(g * 16, 16)]
                sv = src_raw[pl.ds(g * 16, 16)]
                m = (d >= lo) & (d < lo + chunk)
                plsc.store_compressed(lsrc.at[pl.ds(off, 16)], sv, mask=m)
                plsc.store_compressed(ldst.at[pl.ds(off, 16)], d - lo, mask=m)
                return off + jnp.max(plsc.all_reduce_population_count(m))

            m_cnt = lax.fori_loop(0, et // 16, scan, jnp.int32(0))
            nb = (m_cnt + BLK - 1) // BLK

            for g in range(BLK // 16):
                lsrc[pl.ds(m_cnt + g * 16, 16)] = zivec
                ldst[pl.ds(m_cnt + g * 16, 16)] = tvec

            for src_r, dst_r in z_descs():
                pltpu.make_async_copy(src_r, dst_r, zsem).wait()
            plsc.subcore_barrier()

            @pl.when(nb > 0)
            def _():
                g_issue(0, 0)

            def quad(i4, carry):
                for b in range(NBUF):
                    j = i4 * NBUF + b

                    @pl.when(j < nb)
                    def _():
                        jn = j + 1
                        bn = (b + 1) % NBUF

                        @pl.when(jn < nb)
                        def _():
                            @pl.when(jn >= NBUF)
                            def _():
                                s_wait(bn)
                            g_issue(jn, bn)

                        g_wait(b)
                        s_issue(j, b)
                return carry

            lax.fori_loop(0, (nb + NBUF - 1) // NBUF, quad, 0)
            for b in range(NBUF):
                @pl.when(b < nb)
                def _():
                    s_wait(b)
            plsc.subcore_barrier()

            ob = lo + s * wr
            pltpu.sync_copy(acc.at[pl.ds(s * wr, wr)],
                            sums_o.at[pl.ds(ob, wr)])
            if p < P - 1:
                plsc.subcore_barrier()

    f = pl.kernel(
        body,
        out_type=[jax.ShapeDtypeStruct((n_out, HID), F32)],
        mesh=mesh, scratch_types=scratch,
        compiler_params=pltpu.CompilerParams(**_CPARAMS))
    _SEG_CACHE[key] = (f, et, n_out)
    return _SEG_CACHE[key]


_CNT_CACHE = {}


def _make_counts(configs):
    key = tuple(configs)
    if key in _CNT_CACHE:
        return _CNT_CACHE[key]

    geo = []
    for n_dst, n_edges in configs:
        et = _et_of(n_edges)
        chunk = _cdiv(n_dst, NC * 128) * 128
        geo.append((et, chunk))
    et_max = max(g[0] for g in geo)
    a_max = max(g[1] for g in geo) + 128
    CB = 128

    out_type = [jax.ShapeDtypeStruct((NC * g[1], 16), F32) for g in geo]
    scratch = [
        pltpu.VMEM((et_max,), I32),
        pltpu.VMEM((et_max + 16,), I32),
        pltpu.VMEM((CB, 16), F32),
        pltpu.VMEM((CB, 16), F32),
        pltpu.VMEM_SHARED((a_max, 16), F32),
    ]
    scratch += [pltpu.SemaphoreType.DMA] * (NBUF + 1)

    mesh = plsc.VectorSubcoreMesh(**_MESH)
    n_types = len(configs)

    def body(*refs):
        dst_hbms = refs[:n_types]
        outs = refs[n_types:2 * n_types]
        dst_raw, ldst, ones, zcnt, cnt = refs[2 * n_types:2 * n_types + 5]
        sems = refs[2 * n_types + 5:]
        csem = sems[:NBUF]
        zsem = sems[NBUF]

        c = lax.axis_index("c")
        s = lax.axis_index("s")

        zvec = jnp.zeros((16,), F32)
        ovec = jnp.ones((16,), F32)

        def init(r, carry):
            ones[r, :] = ovec
            zcnt[r, :] = zvec
            return carry

        lax.fori_loop(0, CB, init, 0)

        for t in range(n_types):
            et, chunk = geo[t]
            A = chunk + 128
            zr = A // 16
            wr = chunk // 16
            lo = c * chunk

            znf, zrem = divmod(zr, CB)
            zb = s * zr

            def z_descs():
                ds_ = []
                for q in range(znf):
                    ds_.append((zcnt, cnt.at[pl.ds(zb + q * CB, CB)]))
                if zrem:
                    ds_.append((zcnt.at[pl.ds(0, zrem)],
                                cnt.at[pl.ds(zb + znf * CB, zrem)]))
                return ds_

            for src_r, dst_r in z_descs():
                pltpu.async_copy(src_r, dst_r, zsem)

            base = s * et
            pltpu.sync_copy(dst_hbms[t].at[pl.ds(base, et)],
                            dst_raw.at[pl.ds(0, et)])

            tvec = jnp.full((16,), chunk, I32)

            def fill(i, carry):
                ldst[pl.ds(i * 16, 16)] = tvec
                return carry

            lax.fori_loop(0, et // 16 + 1, fill, 0)

            def scan(g, off):
                d = dst_raw[pl.ds(g * 16, 16)]
                m = (d >= lo) & (d < lo + chunk)
                plsc.store_compressed(ldst.at[pl.ds(off, 16)], d - lo, mask=m)
                return off + jnp.max(plsc.all_reduce_population_count(m))

            m_cnt = lax.fori_loop(0, et // 16, scan, jnp.int32(0))
            nb = (m_cnt + CB - 1) // CB

            for src_r, dst_r in z_descs():
                pltpu.make_async_copy(src_r, dst_r, zsem).wait()
            plsc.subcore_barrier()

            def c_issue(j, b):
                pltpu.async_copy(ones, cnt.at[ldst.at[pl.ds(j * CB, CB)]],
                                 csem[b], add=True)

            def c_wait(b):
                pltpu.make_async_copy(
                    ones, cnt.at[ldst.at[pl.ds(0, CB)]], csem[b]).wait()

            def quad(i4, carry):
                for b in range(NBUF):
                    j = i4 * NBUF + b

                    @pl.when(j < nb)
                    def _():
                        @pl.when(j >= NBUF)
                        def _():
                            c_wait(b)
                        c_issue(j, b)
                return carry

            lax.fori_loop(0, (nb + NBUF - 1) // NBUF, quad, 0)
            for b in range(NBUF):
                @pl.when(b < nb)
                def _():
                    c_wait(b)
            plsc.subcore_barrier()

            ob = lo + s * wr
            pltpu.sync_copy(cnt.at[pl.ds(s * wr, wr)],
                            outs[t].at[pl.ds(ob, wr)])
            if t < n_types - 1:
                plsc.subcore_barrier()

    f = pl.kernel(
        body, out_type=out_type, mesh=mesh, scratch_types=scratch,
        compiler_params=pltpu.CompilerParams(**_CPARAMS))
    _CNT_CACHE[key] = f
    return f


def _pad_edges(ei, n_edges_pad):
    e = ei.shape[1]
    pad = n_edges_pad - e
    src = jnp.concatenate([ei[0].astype(I32), jnp.zeros((pad,), I32)])
    dst = jnp.concatenate([ei[1].astype(I32), jnp.full((pad,), -1, I32)])
    return src, dst


def _mm_bias(x, w, b):
    n, kd = x.shape
    m = w.shape[1]
    grid = _cdiv(n, BR)

    def body(x_ref, w_ref, b_ref, o_ref):
        o_ref[...] = (
            jnp.dot(x_ref[...], w_ref[...], preferred_element_type=F32)
            + b_ref[...])

    return pl.pallas_call(
        body,
        grid=(grid,),
        in_specs=[
            pl.BlockSpec((BR, kd), lambda i: (i, 0)),
            pl.BlockSpec((kd, m), lambda i: (0, 0)),
            pl.BlockSpec((1, m), lambda i: (0, 0)),
        ],
        out_specs=pl.BlockSpec((BR, m), lambda i: (i, 0)),
        out_shape=jax.ShapeDtypeStruct((n, m), F32),
    )(x, w, b)


def _combine(h, sums, cnts, wl_stack, wr_sum, blm, g, b):
    n = h.shape[0]
    k = len(sums)
    grid = _cdiv(n, BR)

    def body(*refs):
        h_ref = refs[0]
        s_refs = refs[1:1 + k]
        c_refs = refs[1 + k:1 + 2 * k]
        wl_ref, wr_ref, blm_ref, g_ref, b_ref, o_ref = refs[1 + 2 * k:]
        hv = h_ref[...]
        acc = jnp.dot(hv, wr_ref[...], preferred_element_type=F32)
        for i in range(k):
            cntv = c_refs[i][...][:, 0:1]
            recip = 1.0 / jnp.maximum(cntv, 1.0)
            acc = acc + jnp.dot(s_refs[i][...] * recip, wl_ref[i],
                                preferred_element_type=F32)
        x = acc * (1.0 / k) + blm_ref[...]
        mu = jnp.mean(x, axis=-1, keepdims=True)
        var = jnp.mean((x - mu) ** 2, axis=-1, keepdims=True)
        xn = (x - mu) * lax.rsqrt(var + 1e-5) * g_ref[...] + b_ref[...]
        o_ref[...] = xn + hv

    in_specs = [pl.BlockSpec((BR, HID), lambda i: (i, 0))]
    in_specs += [pl.BlockSpec((BR, HID), lambda i: (i, 0))] * k
    in_specs += [pl.BlockSpec((BR, 16), lambda i: (i, 0))] * k
    in_specs += [
        pl.BlockSpec((k, HID, HID), lambda i: (0, 0, 0)),
        pl.BlockSpec((HID, HID), lambda i: (0, 0)),
        pl.BlockSpec((1, HID), lambda i: (0, 0)),
        pl.BlockSpec((1, HID), lambda i: (0, 0)),
        pl.BlockSpec((1, HID), lambda i: (0, 0)),
    ]
    return pl.pallas_call(
        body,
        grid=(grid,),
        in_specs=in_specs,
        out_specs=pl.BlockSpec((BR, HID), lambda i: (i, 0)),
        out_shape=jax.ShapeDtypeStruct((n, HID), F32),
    )(h, *sums, *cnts, wl_stack, wr_sum, blm, g, b)


def kernel(x_occ, x_chord, x_sec, ei_next, ei_prev, ei_inst, ei_inst_rev,
           ei_in_sec, ei_sec_rev, ei_next_sec, Wp_occ, bp_occ, Wp_chord,
           bp_chord, Wp_sec, bp_sec, Wl, bl, Wr, ln_g, ln_b, Wc, bc):
    n = {'occ': x_occ.shape[0], 'chord': x_chord.shape[0],
         'sec': x_sec.shape[0]}
    meta = [('occ', 'occ'), ('occ', 'occ'), ('occ', 'chord'),
            ('chord', 'occ'), ('occ', 'sec'), ('sec', 'occ'), ('sec', 'sec')]
    eis = [ei_next, ei_prev, ei_inst, ei_inst_rev, ei_in_sec, ei_sec_rev,
           ei_next_sec]
    incoming = {'occ': [0, 1, 3, 5], 'chord': [2], 'sec': [4, 6]}
    num_layers = Wl.shape[0]

    h = {'occ': _mm_bias(x_occ, Wp_occ, bp_occ[None]),
         'chord': _mm_bias(x_chord, Wp_chord, bp_chord[None]),
         'sec': _mm_bias(x_sec, Wp_sec, bp_sec[None])}

    seg = []
    eip = []
    for i, (st, dt) in enumerate(meta):
        f, et, n_out = _make_seg_sum(n[st], n[dt], eis[i].shape[1])
        seg.append(f)
        eip.append(_pad_edges(eis[i], NS * et))

    cfg = tuple((n[dt], eis[i].shape[1]) for i, (st, dt) in enumerate(meta))
    fcnt = _make_counts(cfg)
    cnts = fcnt(*[eip[i][1] for i in range(len(meta))])
    cnts = list(cnts) if isinstance(cnts, (tuple, list)) else [cnts]

    for l in range(num_layers):
        sums = {}
        for i, (st, dt) in enumerate(meta):
            out = seg[i](h[st], eip[i][0], eip[i][1])
            sums[i] = out[0] if isinstance(out, (tuple, list)) else out
        h_new = {}
        for nt in ('occ', 'chord', 'sec'):
            idxs = incoming[nt]
            k = len(idxs)
            wl_stack = jnp.stack([Wl[l, i] for i in idxs])
            wr_sum = sum(Wr[l, i] for i in idxs)
            blm = (sum(bl[l, i] for i in idxs) / k)[None]
            h_new[nt] = _combine(
                h[nt], [sums[i] for i in idxs], [cnts[i] for i in idxs],
                wl_stack, wr_sum, blm, ln_g[l][None], ln_b[l][None])
        h = h_new

    return _mm_bias(h['occ'], Wc, bc[None])
